# Initial kernel scaffold; baseline (speedup 1.0000x reference)
import jax
import jax.numpy as jnp
from jax.experimental import pallas as pl


def kernel(gate_type, edge_index, edge_src_idx, edge_dst_idx, edge_reversed, emb, W1_0, W2_0, b2_0, W1_1, W2_1, b2_1, W1_2, W2_2, b2_2, W1_3, W2_3, b2_3, W1_4, W2_4, b2_4, L1W, L1b, L2W, L2b):
    raise NotImplementedError("write your pallas kernel here")


# speedup vs baseline: 3.5586x; 3.5586x over previous
import functools

import jax
import jax.numpy as jnp
from jax import lax
from jax.experimental import pallas as pl
from jax.experimental.pallas import tpu as pltpu
from jax.experimental.pallas import tpu_sc as plsc

N = 10000
E = 320000
F = 128
D = 32
NC, NS = 2, 16
RW = 128
ROWS = E // RW
ROWS_PER_SC = ROWS // NC
NPS = N // NS
NB = 400
GRID_N = N // NB
EB = 6400
GRID_E = E // EB

_HI = lax.Precision.HIGHEST


def _dot(a, b):
  return jnp.dot(a, b, preferred_element_type=jnp.float32, precision=_HI)


def _sc_edge_body(with_deg, *refs):
  if with_deg:
    (p_hbm, wq_hbm, src_hbm, dst_hbm, accs_hbm, deg0_hbm, deg1_hbm,
     idx_v, dst_v, rows_v, wq_v, ones_v, vz_v, acc_sh, deg_sh, sem) = refs
  else:
    (p_hbm, wq_hbm, src_hbm, dst_hbm, accs_hbm,
     idx_v, dst_v, rows_v, wq_v, acc_sh, sem) = refs

  cid = lax.axis_index("c")
  sid = lax.axis_index("s")
  zv = jnp.zeros((16,), jnp.float32)

  def zero_rows(e, c):
    rows_v[e, pl.ds(0, 16)] = zv
    rows_v[e, pl.ds(16, 16)] = zv
    return c
  lax.fori_loop(0, RW, zero_rows, 0)
  zbase = sid * 640

  @pl.when(sid < NS - 1)
  def _():
    for off in (0, 128, 256, 384, 512):
      pltpu.sync_copy(rows_v, acc_sh.at[pl.ds(zbase + off, 128)])

  @pl.when(sid == NS - 1)
  def _():
    for off in (0, 128, 256):
      pltpu.sync_copy(rows_v, acc_sh.at[pl.ds(zbase + off, 128)])
    pltpu.sync_copy(rows_v.at[pl.ds(0, 16)], acc_sh.at[pl.ds(zbase + 384, 16)])

  if with_deg:
    def zero_vz(i, c):
      vz_v[pl.ds(i * 16, 16)] = zv
      return c
    lax.fori_loop(0, 40, zero_vz, 0)
    dzb = sid * 640

    @pl.when(sid < NS - 1)
    def _():
      pltpu.sync_copy(vz_v, deg_sh.at[pl.ds(dzb, 640)])

    @pl.when(sid == NS - 1)
    def _():
      pltpu.sync_copy(vz_v.at[pl.ds(0, 400)], deg_sh.at[pl.ds(dzb, 400)])

    ov = jnp.ones((16,), jnp.float32)
    def fill_ones(i, c):
      ones_v[pl.ds(i * 16, 16)] = ov
      return c
    lax.fori_loop(0, 8, fill_ones, 0)

  plsc.subcore_barrier()

  start = cid * ROWS_PER_SC + sid * 78 + jnp.minimum(sid, 2)
  nrows = jnp.where(sid < 2, 79, 78)

  def row_body(k, c):
    row = start + k
    pltpu.sync_copy(src_hbm.at[row], idx_v)
    pltpu.sync_copy(dst_hbm.at[row], dst_v)
    pltpu.async_copy(p_hbm.at[idx_v], rows_v, sem).wait()
    pltpu.sync_copy(wq_hbm.at[row], wq_v)

    def comp(e, cc):
      a = rows_v[e, pl.ds(0, 16)] + wq_v[e, pl.ds(0, 16)]
      rows_v[e, pl.ds(0, 16)] = jnp.maximum(a, a * 0.01)
      b = rows_v[e, pl.ds(16, 16)] + wq_v[e, pl.ds(16, 16)]
      rows_v[e, pl.ds(16, 16)] = jnp.maximum(b, b * 0.01)
      return cc
    lax.fori_loop(0, RW, comp, 0)

    pltpu.sync_copy(rows_v, acc_sh.at[dst_v], add=True)
    if with_deg:
      pltpu.sync_copy(ones_v, deg_sh.at[dst_v], add=True)
    return c
  lax.fori_loop(0, nrows, row_body, 0)

  plsc.subcore_barrier()

  @pl.when(sid < NS - 1)
  def _():
    pltpu.sync_copy(acc_sh.at[pl.ds(zbase, 640)],
                    accs_hbm.at[cid, pl.ds(zbase, 640)])

  @pl.when(sid == NS - 1)
  def _():
    pltpu.sync_copy(acc_sh.at[pl.ds(zbase, 400)],
                    accs_hbm.at[cid, pl.ds(zbase, 400)])

  if with_deg:
    dzb = sid * 640
    ln_tail = 400

    @pl.when((sid < NS - 1) & (cid == 0))
    def _():
      pltpu.sync_copy(deg_sh.at[pl.ds(dzb, 640)], deg0_hbm.at[pl.ds(dzb, 640)])

    @pl.when((sid == NS - 1) & (cid == 0))
    def _():
      pltpu.sync_copy(deg_sh.at[pl.ds(dzb, ln_tail)],
                      deg0_hbm.at[pl.ds(dzb, ln_tail)])

    @pl.when((sid < NS - 1) & (cid == 1))
    def _():
      pltpu.sync_copy(deg_sh.at[pl.ds(dzb, 640)], deg1_hbm.at[pl.ds(dzb, 640)])

    @pl.when((sid == NS - 1) & (cid == 1))
    def _():
      pltpu.sync_copy(deg_sh.at[pl.ds(dzb, ln_tail)],
                      deg1_hbm.at[pl.ds(dzb, ln_tail)])


_SC_MESH = plsc.VectorSubcoreMesh(
    core_axis_name="c", subcore_axis_name="s", num_cores=NC, num_subcores=NS)

_SC_PARAMS = pltpu.CompilerParams(use_tc_tiling_on_sc=False)

_sc_edge_deg = pl.kernel(
    functools.partial(_sc_edge_body, True),
    out_type=(jax.ShapeDtypeStruct((NC, N, D), jnp.float32),
              jax.ShapeDtypeStruct((N,), jnp.float32),
              jax.ShapeDtypeStruct((N,), jnp.float32)),
    mesh=_SC_MESH,
    scratch_types=[
        pltpu.VMEM((RW,), jnp.int32),
        pltpu.VMEM((RW,), jnp.int32),
        pltpu.VMEM((RW, D), jnp.float32),
        pltpu.VMEM((RW, D), jnp.float32),
        pltpu.VMEM((RW,), jnp.float32),
        pltpu.VMEM((640,), jnp.float32),
        pltpu.VMEM_SHARED((N, D), jnp.float32),
        pltpu.VMEM_SHARED((N,), jnp.float32),
        pltpu.SemaphoreType.DMA,
    ],
    compiler_params=_SC_PARAMS,
    name="sc_edge_deg",
)

_sc_edge = pl.kernel(
    functools.partial(_sc_edge_body, False),
    out_type=jax.ShapeDtypeStruct((NC, N, D), jnp.float32),
    mesh=_SC_MESH,
    scratch_types=[
        pltpu.VMEM((RW,), jnp.int32),
        pltpu.VMEM((RW,), jnp.int32),
        pltpu.VMEM((RW, D), jnp.float32),
        pltpu.VMEM((RW, D), jnp.float32),
        pltpu.VMEM_SHARED((N, D), jnp.float32),
        pltpu.SemaphoreType.DMA,
    ],
    compiler_params=_SC_PARAMS,
    name="sc_edge",
)


def _prep_body(gate_ref, emb_ref, w1aT_ref, h0_ref, p0_ref):
  g = gate_ref[...]
  iota = lax.broadcasted_iota(jnp.int32, (NB, F), 1)
  oh = jnp.where(g == iota, 1.0, 0.0).astype(jnp.float32)
  h0 = _dot(oh, emb_ref[...])
  h0_ref[...] = h0
  p0_ref[...] = _dot(h0, w1aT_ref[...])


_prep = pl.pallas_call(
    _prep_body,
    grid=(GRID_N,),
    in_specs=[
        pl.BlockSpec((NB, 1), lambda i: (i, 0)),
        pl.BlockSpec((F, F), lambda i: (0, 0)),
        pl.BlockSpec((F, D), lambda i: (0, 0)),
    ],
    out_specs=(pl.BlockSpec((NB, F), lambda i: (i, 0)),
               pl.BlockSpec((NB, D), lambda i: (i, 0))),
    out_shape=(jax.ShapeDtypeStruct((N, F), jnp.float32),
               jax.ShapeDtypeStruct((N, D), jnp.float32)),
)


def _wq_body(w8_ref, w1bT8_ref, wq_ref):
  wq_ref[...] = _dot(w8_ref[...], w1bT8_ref[...])


_wq = pl.pallas_call(
    _wq_body,
    grid=(GRID_E,),
    in_specs=[
        pl.BlockSpec((EB, 8), lambda i: (i, 0)),
        pl.BlockSpec((8, D), lambda i: (0, 0)),
    ],
    out_specs=pl.BlockSpec((EB, D), lambda i: (i, 0)),
    out_shape=jax.ShapeDtypeStruct((E, D), jnp.float32),
)


def _combine(a0, a1, d0, d1):
  deg = jnp.maximum(d0 + d1, 1.0)
  return (a0 + a1) / deg


def _layer_body(h_ref, a0_ref, a1_ref, d0_ref, d1_ref, w2aT_ref, w2bT_ref,
                b2_ref, w1aT_ref, hout_ref, pout_ref):
  hN = _combine(a0_ref[...], a1_ref[...], d0_ref[...], d1_ref[...])
  z = _dot(h_ref[...], w2aT_ref[...]) + _dot(hN, w2bT_ref[...]) + b2_ref[...]
  hn = jnp.maximum(z, 0.0)
  hout_ref[...] = hn
  pout_ref[...] = _dot(hn, w1aT_ref[...])


_layer = pl.pallas_call(
    _layer_body,
    grid=(GRID_N,),
    in_specs=[
        pl.BlockSpec((NB, F), lambda i: (i, 0)),
        pl.BlockSpec((NB, D), lambda i: (i, 0)),
        pl.BlockSpec((NB, D), lambda i: (i, 0)),
        pl.BlockSpec((NB, 1), lambda i: (i, 0)),
        pl.BlockSpec((NB, 1), lambda i: (i, 0)),
        pl.BlockSpec((F, F), lambda i: (0, 0)),
        pl.BlockSpec((D, F), lambda i: (0, 0)),
        pl.BlockSpec((1, F), lambda i: (0, 0)),
        pl.BlockSpec((F, D), lambda i: (0, 0)),
    ],
    out_specs=(pl.BlockSpec((NB, F), lambda i: (i, 0)),
               pl.BlockSpec((NB, D), lambda i: (i, 0))),
    out_shape=(jax.ShapeDtypeStruct((N, F), jnp.float32),
               jax.ShapeDtypeStruct((N, D), jnp.float32)),
)


def _final_body(h_ref, a0_ref, a1_ref, d0_ref, d1_ref, w2aT_ref, w2bT_ref,
                b2_ref, l1wT_ref, l1b_ref, l2wT_ref, l2b_ref, out_ref):
  hN = _combine(a0_ref[...], a1_ref[...], d0_ref[...], d1_ref[...])
  z = _dot(h_ref[...], w2aT_ref[...]) + _dot(hN, w2bT_ref[...]) + b2_ref[...]
  hn = jnp.maximum(z, 0.0)
  h1 = jnp.maximum(_dot(hn, l1wT_ref[...]) + l1b_ref[...], 0.0)
  out_ref[...] = _dot(h1, l2wT_ref[...]) + l2b_ref[...]


_final = pl.pallas_call(
    _final_body,
    grid=(GRID_N,),
    in_specs=[
        pl.BlockSpec((NB, F), lambda i: (i, 0)),
        pl.BlockSpec((NB, D), lambda i: (i, 0)),
        pl.BlockSpec((NB, D), lambda i: (i, 0)),
        pl.BlockSpec((NB, 1), lambda i: (i, 0)),
        pl.BlockSpec((NB, 1), lambda i: (i, 0)),
        pl.BlockSpec((F, F), lambda i: (0, 0)),
        pl.BlockSpec((D, F), lambda i: (0, 0)),
        pl.BlockSpec((1, F), lambda i: (0, 0)),
        pl.BlockSpec((F, F), lambda i: (0, 0)),
        pl.BlockSpec((1, F), lambda i: (0, 0)),
        pl.BlockSpec((F, D), lambda i: (0, 0)),
        pl.BlockSpec((1, D), lambda i: (0, 0)),
    ],
    out_specs=pl.BlockSpec((NB, D), lambda i: (i, 0)),
    out_shape=jax.ShapeDtypeStruct((N, D), jnp.float32),
)


def kernel(gate_type, edge_index, edge_src_idx, edge_dst_idx, edge_reversed,
           emb, W1_0, W2_0, b2_0, W1_1, W2_1, b2_1, W1_2, W2_2, b2_2,
           W1_3, W2_3, b2_3, W1_4, W2_4, b2_4, L1W, L1b, L2W, L2b):
  f32 = jnp.float32
  gate2d = gate_type.astype(jnp.int32).reshape(N, 1)
  src2d = edge_index[0].astype(jnp.int32).reshape(ROWS, RW)
  dst2d = edge_index[1].astype(jnp.int32).reshape(ROWS, RW)
  w8 = jnp.concatenate(
      [edge_src_idx[:, None], edge_dst_idx[:, None], edge_reversed[:, None],
       jnp.zeros((E, 5), f32)], axis=1)

  convs = [(W1_0, W2_0, b2_0), (W1_1, W2_1, b2_1), (W1_2, W2_2, b2_2),
           (W1_3, W2_3, b2_3), (W1_4, W2_4, b2_4)]
  W1aT, W1bT8, W2aT, W2bT, b2r = [], [], [], [], []
  for W1, W2, b2 in convs:
    W1aT.append(W1[:, :F].T)
    W1bT8.append(jnp.concatenate([W1[:, F:F + 3].T, jnp.zeros((5, D), f32)],
                                 axis=0))
    W2aT.append(W2[:, :F].T)
    W2bT.append(W2[:, F:].T)
    b2r.append(b2.reshape(1, F))

  h, p = _prep(gate2d, emb, W1aT[0])

  out = None
  d0 = d1 = None
  for l in range(5):
    wq = _wq(w8, W1bT8[l]).reshape(ROWS, RW, D)
    if l == 0:
      accs, deg0, deg1 = _sc_edge_deg(p, wq, src2d, dst2d)
      d0 = deg0.reshape(N, 1)
      d1 = deg1.reshape(N, 1)
    else:
      accs = _sc_edge(p, wq, src2d, dst2d)
    a0, a1 = accs[0], accs[1]
    if l < 4:
      h, p = _layer(h, a0, a1, d0, d1, W2aT[l], W2bT[l], b2r[l], W1aT[l + 1])
    else:
      out = _final(h, a0, a1, d0, d1, W2aT[4], W2bT[4], b2r[4],
                   L1W.T, L1b.reshape(1, F), L2W.T, L2b.reshape(1, D))
  return out


# speedup vs baseline: 4.3904x; 1.2337x over previous
import functools

import jax
import jax.numpy as jnp
from jax import lax
from jax.experimental import pallas as pl
from jax.experimental.pallas import tpu as pltpu
from jax.experimental.pallas import tpu_sc as plsc

N = 10000
E = 320000
F = 128
D = 32
NC, NS = 2, 16
RW = 128
ROWS = E // RW
ROWS_PER_SC = ROWS // NC
NPS = N // NS
NB = 400
GRID_N = N // NB
EB = 6400
GRID_E = E // EB

_HI = lax.Precision.HIGHEST


def _dot(a, b):
  return jnp.dot(a, b, preferred_element_type=jnp.float32, precision=_HI)


G = 6
NBLK = 13
RPT = G * NBLK
GE = G * RW


def _sc_edge_body(with_deg, *refs):
  if with_deg:
    (p_hbm, wq_hbm, src_hbm, dst_hbm, accs_hbm, deg0_hbm, deg1_hbm,
     idx2, dst2, dsts, rows2, wq2, ones_v, vz_v, acc_sh, deg_sh,
     sem_ld0, sem_ld1, sem_g0, sem_g1, sem_s0, sem_s1) = refs
  else:
    (p_hbm, wq_hbm, src_hbm, dst_hbm, accs_hbm,
     idx2, dst2, dsts, rows2, wq2, acc_sh,
     sem_ld0, sem_ld1, sem_g0, sem_g1, sem_s0, sem_s1) = refs
  sem_ld = (sem_ld0, sem_ld1)
  sem_g = (sem_g0, sem_g1)
  sem_s = (sem_s0, sem_s1)

  cid = lax.axis_index("c")
  sid = lax.axis_index("s")
  wid = sid * NC + cid
  zv = jnp.zeros((16,), jnp.float32)

  def zero_rows(e, c):
    rows2[0, e, pl.ds(0, 16)] = zv
    rows2[0, e, pl.ds(16, 16)] = zv
    return c
  lax.fori_loop(0, RW, zero_rows, 0)
  rows_v = rows2.at[0, pl.ds(0, RW)]
  zbase = sid * 640

  @pl.when(sid < NS - 1)
  def _():
    for off in (0, 128, 256, 384, 512):
      pltpu.sync_copy(rows_v, acc_sh.at[pl.ds(zbase + off, 128)])

  @pl.when(sid == NS - 1)
  def _():
    for off in (0, 128, 256):
      pltpu.sync_copy(rows_v, acc_sh.at[pl.ds(zbase + off, 128)])
    pltpu.sync_copy(rows2.at[0, pl.ds(0, 16)],
                    acc_sh.at[pl.ds(zbase + 384, 16)])

  if with_deg:
    def zero_vz(i, c):
      vz_v[pl.ds(i * 16, 16)] = zv
      return c
    lax.fori_loop(0, 40, zero_vz, 0)
    dzb = sid * 640

    @pl.when(sid < NS - 1)
    def _():
      pltpu.sync_copy(vz_v, deg_sh.at[pl.ds(dzb, 640)])

    @pl.when(sid == NS - 1)
    def _():
      pltpu.sync_copy(vz_v.at[pl.ds(0, 400)], deg_sh.at[pl.ds(dzb, 400)])

    ov = jnp.ones((16,), jnp.float32)
    def fill_ones(i, c):
      ones_v[pl.ds(i * 16, 16)] = ov
      return c
    lax.fori_loop(0, 8, fill_ones, 0)

  plsc.subcore_barrier()

  base_row = wid * RPT

  def issue_loads(b, buf):
    r0 = base_row + b * G
    return [
        pltpu.async_copy(src_hbm.at[pl.ds(r0, G)], idx2.at[buf], sem_ld[buf]),
        pltpu.async_copy(dst_hbm.at[pl.ds(r0, G)], dst2.at[buf], sem_ld[buf]),
        pltpu.async_copy(wq_hbm.at[pl.ds(r0 * RW, GE)], wq2.at[buf],
                         sem_ld[buf]),
    ]

  def issue_gathers(buf):
    return [
        pltpu.async_copy(p_hbm.at[idx2.at[buf, j]],
                         rows2.at[buf, pl.ds(j * RW, RW)], sem_g[buf])
        for j in range(G)
    ]

  def compute_block(buf, n_edges):
    def comp(e, cc):
      a = rows2[buf, e, pl.ds(0, 16)] + wq2[buf, e, pl.ds(0, 16)]
      rows2[buf, e, pl.ds(0, 16)] = jnp.maximum(a, a * 0.01)
      bb = rows2[buf, e, pl.ds(16, 16)] + wq2[buf, e, pl.ds(16, 16)]
      rows2[buf, e, pl.ds(16, 16)] = jnp.maximum(bb, bb * 0.01)
      return cc
    lax.fori_loop(0, n_edges, comp, 0)

  def issue_scatters(buf):
    def cp(i, c):
      j = i // 8
      k = (i % 8) * 16
      dsts[buf, j, pl.ds(k, 16)] = dst2[buf, j, pl.ds(k, 16)]
      return c
    lax.fori_loop(0, G * 8, cp, 0)
    ds_ = [
        pltpu.async_copy(rows2.at[buf, pl.ds(j * RW, RW)],
                         acc_sh.at[dsts.at[buf, j]], sem_s[buf], add=True)
        for j in range(G)
    ]
    if with_deg:
      ds_ += [
          pltpu.async_copy(ones_v, deg_sh.at[dsts.at[buf, j]], sem_s[buf],
                           add=True)
          for j in range(G)
      ]
    return ds_

  ld = [None, None]
  g = [None, None]
  s = [None, None]
  ld[0] = issue_loads(0, 0)
  for d in ld[0]:
    d.wait()
  g[0] = issue_gathers(0)
  ld[1] = issue_loads(1, 1)
  for b in range(NBLK):
    buf = b % 2
    obuf = 1 - buf
    for d in g[buf]:
      d.wait()
    if b + 1 < NBLK:
      for d in ld[obuf]:
        d.wait()
      if b >= 1:
        for d in s[obuf]:
          d.wait()
      g[obuf] = issue_gathers(obuf)
    compute_block(buf, GE)
    s[buf] = issue_scatters(buf)
    if b + 2 < NBLK:
      ld[buf] = issue_loads(b + 2, buf)
  for d in s[0] + s[1]:
    d.wait()

  @pl.when(wid < ROWS - 32 * RPT)
  def _():
    trow = 32 * RPT + wid
    pltpu.async_copy(src_hbm.at[pl.ds(trow, 1)], idx2.at[0, pl.ds(0, 1)],
                     sem_ld[0]).wait()
    pltpu.async_copy(dst_hbm.at[pl.ds(trow, 1)], dst2.at[0, pl.ds(0, 1)],
                     sem_ld[0]).wait()
    pltpu.async_copy(wq_hbm.at[pl.ds(trow * RW, RW)],
                     wq2.at[0, pl.ds(0, RW)], sem_ld[0]).wait()
    pltpu.async_copy(p_hbm.at[idx2.at[0, 0]], rows2.at[0, pl.ds(0, RW)],
                     sem_g[0]).wait()
    compute_block(0, RW)
    pltpu.async_copy(rows2.at[0, pl.ds(0, RW)], acc_sh.at[dst2.at[0, 0]],
                     sem_s[0], add=True).wait()
    if with_deg:
      pltpu.async_copy(ones_v, deg_sh.at[dst2.at[0, 0]], sem_s[0],
                       add=True).wait()

  plsc.subcore_barrier()

  @pl.when(sid < NS - 1)
  def _():
    pltpu.sync_copy(acc_sh.at[pl.ds(zbase, 640)],
                    accs_hbm.at[cid, pl.ds(zbase, 640)])

  @pl.when(sid == NS - 1)
  def _():
    pltpu.sync_copy(acc_sh.at[pl.ds(zbase, 400)],
                    accs_hbm.at[cid, pl.ds(zbase, 400)])

  if with_deg:
    dzb = sid * 640
    ln_tail = 400

    @pl.when((sid < NS - 1) & (cid == 0))
    def _():
      pltpu.sync_copy(deg_sh.at[pl.ds(dzb, 640)], deg0_hbm.at[pl.ds(dzb, 640)])

    @pl.when((sid == NS - 1) & (cid == 0))
    def _():
      pltpu.sync_copy(deg_sh.at[pl.ds(dzb, ln_tail)],
                      deg0_hbm.at[pl.ds(dzb, ln_tail)])

    @pl.when((sid < NS - 1) & (cid == 1))
    def _():
      pltpu.sync_copy(deg_sh.at[pl.ds(dzb, 640)], deg1_hbm.at[pl.ds(dzb, 640)])

    @pl.when((sid == NS - 1) & (cid == 1))
    def _():
      pltpu.sync_copy(deg_sh.at[pl.ds(dzb, ln_tail)],
                      deg1_hbm.at[pl.ds(dzb, ln_tail)])


_SC_MESH = plsc.VectorSubcoreMesh(
    core_axis_name="c", subcore_axis_name="s", num_cores=NC, num_subcores=NS)

_SC_PARAMS = pltpu.CompilerParams(use_tc_tiling_on_sc=False)

_sc_edge_deg = pl.kernel(
    functools.partial(_sc_edge_body, True),
    out_type=(jax.ShapeDtypeStruct((NC, N, D), jnp.float32),
              jax.ShapeDtypeStruct((N,), jnp.float32),
              jax.ShapeDtypeStruct((N,), jnp.float32)),
    mesh=_SC_MESH,
    scratch_types=[
        pltpu.VMEM((2, G, RW), jnp.int32),
        pltpu.VMEM((2, G, RW), jnp.int32),
        pltpu.VMEM((2, G, RW), jnp.int32),
        pltpu.VMEM((2, GE, D), jnp.float32),
        pltpu.VMEM((2, GE, D), jnp.float32),
        pltpu.VMEM((RW,), jnp.float32),
        pltpu.VMEM((640,), jnp.float32),
        pltpu.VMEM_SHARED((N, D), jnp.float32),
        pltpu.VMEM_SHARED((N,), jnp.float32),
        pltpu.SemaphoreType.DMA,
        pltpu.SemaphoreType.DMA,
        pltpu.SemaphoreType.DMA,
        pltpu.SemaphoreType.DMA,
        pltpu.SemaphoreType.DMA,
        pltpu.SemaphoreType.DMA,
    ],
    compiler_params=_SC_PARAMS,
    name="sc_edge_deg",
)

_sc_edge = pl.kernel(
    functools.partial(_sc_edge_body, False),
    out_type=jax.ShapeDtypeStruct((NC, N, D), jnp.float32),
    mesh=_SC_MESH,
    scratch_types=[
        pltpu.VMEM((2, G, RW), jnp.int32),
        pltpu.VMEM((2, G, RW), jnp.int32),
        pltpu.VMEM((2, G, RW), jnp.int32),
        pltpu.VMEM((2, GE, D), jnp.float32),
        pltpu.VMEM((2, GE, D), jnp.float32),
        pltpu.VMEM_SHARED((N, D), jnp.float32),
        pltpu.SemaphoreType.DMA,
        pltpu.SemaphoreType.DMA,
        pltpu.SemaphoreType.DMA,
        pltpu.SemaphoreType.DMA,
        pltpu.SemaphoreType.DMA,
        pltpu.SemaphoreType.DMA,
    ],
    compiler_params=_SC_PARAMS,
    name="sc_edge",
)


def _prep_body(gate_ref, emb_ref, w1aT_ref, h0_ref, p0_ref):
  g = gate_ref[...]
  iota = lax.broadcasted_iota(jnp.int32, (NB, F), 1)
  oh = jnp.where(g == iota, 1.0, 0.0).astype(jnp.float32)
  h0 = _dot(oh, emb_ref[...])
  h0_ref[...] = h0
  p0_ref[...] = _dot(h0, w1aT_ref[...])


_prep = pl.pallas_call(
    _prep_body,
    grid=(GRID_N,),
    in_specs=[
        pl.BlockSpec((NB, 1), lambda i: (i, 0)),
        pl.BlockSpec((F, F), lambda i: (0, 0)),
        pl.BlockSpec((F, D), lambda i: (0, 0)),
    ],
    out_specs=(pl.BlockSpec((NB, F), lambda i: (i, 0)),
               pl.BlockSpec((NB, D), lambda i: (i, 0))),
    out_shape=(jax.ShapeDtypeStruct((N, F), jnp.float32),
               jax.ShapeDtypeStruct((N, D), jnp.float32)),
)


def _wq_body(w8_ref, w1bT8_ref, wq_ref):
  wq_ref[...] = _dot(w8_ref[...], w1bT8_ref[...])


_wq = pl.pallas_call(
    _wq_body,
    grid=(GRID_E,),
    in_specs=[
        pl.BlockSpec((EB, 8), lambda i: (i, 0)),
        pl.BlockSpec((8, D), lambda i: (0, 0)),
    ],
    out_specs=pl.BlockSpec((EB, D), lambda i: (i, 0)),
    out_shape=jax.ShapeDtypeStruct((E, D), jnp.float32),
)


def _combine(a0, a1, d0, d1):
  deg = jnp.maximum(d0 + d1, 1.0)
  return (a0 + a1) / deg


def _layer_body(h_ref, a0_ref, a1_ref, d0_ref, d1_ref, w2aT_ref, w2bT_ref,
                b2_ref, w1aT_ref, hout_ref, pout_ref):
  hN = _combine(a0_ref[...], a1_ref[...], d0_ref[...], d1_ref[...])
  z = _dot(h_ref[...], w2aT_ref[...]) + _dot(hN, w2bT_ref[...]) + b2_ref[...]
  hn = jnp.maximum(z, 0.0)
  hout_ref[...] = hn
  pout_ref[...] = _dot(hn, w1aT_ref[...])


_layer = pl.pallas_call(
    _layer_body,
    grid=(GRID_N,),
    in_specs=[
        pl.BlockSpec((NB, F), lambda i: (i, 0)),
        pl.BlockSpec((NB, D), lambda i: (i, 0)),
        pl.BlockSpec((NB, D), lambda i: (i, 0)),
        pl.BlockSpec((NB, 1), lambda i: (i, 0)),
        pl.BlockSpec((NB, 1), lambda i: (i, 0)),
        pl.BlockSpec((F, F), lambda i: (0, 0)),
        pl.BlockSpec((D, F), lambda i: (0, 0)),
        pl.BlockSpec((1, F), lambda i: (0, 0)),
        pl.BlockSpec((F, D), lambda i: (0, 0)),
    ],
    out_specs=(pl.BlockSpec((NB, F), lambda i: (i, 0)),
               pl.BlockSpec((NB, D), lambda i: (i, 0))),
    out_shape=(jax.ShapeDtypeStruct((N, F), jnp.float32),
               jax.ShapeDtypeStruct((N, D), jnp.float32)),
)


def _final_body(h_ref, a0_ref, a1_ref, d0_ref, d1_ref, w2aT_ref, w2bT_ref,
                b2_ref, l1wT_ref, l1b_ref, l2wT_ref, l2b_ref, out_ref):
  hN = _combine(a0_ref[...], a1_ref[...], d0_ref[...], d1_ref[...])
  z = _dot(h_ref[...], w2aT_ref[...]) + _dot(hN, w2bT_ref[...]) + b2_ref[...]
  hn = jnp.maximum(z, 0.0)
  h1 = jnp.maximum(_dot(hn, l1wT_ref[...]) + l1b_ref[...], 0.0)
  out_ref[...] = _dot(h1, l2wT_ref[...]) + l2b_ref[...]


_final = pl.pallas_call(
    _final_body,
    grid=(GRID_N,),
    in_specs=[
        pl.BlockSpec((NB, F), lambda i: (i, 0)),
        pl.BlockSpec((NB, D), lambda i: (i, 0)),
        pl.BlockSpec((NB, D), lambda i: (i, 0)),
        pl.BlockSpec((NB, 1), lambda i: (i, 0)),
        pl.BlockSpec((NB, 1), lambda i: (i, 0)),
        pl.BlockSpec((F, F), lambda i: (0, 0)),
        pl.BlockSpec((D, F), lambda i: (0, 0)),
        pl.BlockSpec((1, F), lambda i: (0, 0)),
        pl.BlockSpec((F, F), lambda i: (0, 0)),
        pl.BlockSpec((1, F), lambda i: (0, 0)),
        pl.BlockSpec((F, D), lambda i: (0, 0)),
        pl.BlockSpec((1, D), lambda i: (0, 0)),
    ],
    out_specs=pl.BlockSpec((NB, D), lambda i: (i, 0)),
    out_shape=jax.ShapeDtypeStruct((N, D), jnp.float32),
)


def kernel(gate_type, edge_index, edge_src_idx, edge_dst_idx, edge_reversed,
           emb, W1_0, W2_0, b2_0, W1_1, W2_1, b2_1, W1_2, W2_2, b2_2,
           W1_3, W2_3, b2_3, W1_4, W2_4, b2_4, L1W, L1b, L2W, L2b):
  f32 = jnp.float32
  gate2d = gate_type.astype(jnp.int32).reshape(N, 1)
  src2d = edge_index[0].astype(jnp.int32).reshape(ROWS, RW)
  dst2d = edge_index[1].astype(jnp.int32).reshape(ROWS, RW)
  w8 = jnp.concatenate(
      [edge_src_idx[:, None], edge_dst_idx[:, None], edge_reversed[:, None],
       jnp.zeros((E, 5), f32)], axis=1)

  convs = [(W1_0, W2_0, b2_0), (W1_1, W2_1, b2_1), (W1_2, W2_2, b2_2),
           (W1_3, W2_3, b2_3), (W1_4, W2_4, b2_4)]
  W1aT, W1bT8, W2aT, W2bT, b2r = [], [], [], [], []
  for W1, W2, b2 in convs:
    W1aT.append(W1[:, :F].T)
    W1bT8.append(jnp.concatenate([W1[:, F:F + 3].T, jnp.zeros((5, D), f32)],
                                 axis=0))
    W2aT.append(W2[:, :F].T)
    W2bT.append(W2[:, F:].T)
    b2r.append(b2.reshape(1, F))

  h, p = _prep(gate2d, emb, W1aT[0])

  out = None
  d0 = d1 = None
  for l in range(5):
    wq = _wq(w8, W1bT8[l])
    if l == 0:
      accs, deg0, deg1 = _sc_edge_deg(p, wq, src2d, dst2d)
      d0 = deg0.reshape(N, 1)
      d1 = deg1.reshape(N, 1)
    else:
      accs = _sc_edge(p, wq, src2d, dst2d)
    a0, a1 = accs[0], accs[1]
    if l < 4:
      h, p = _layer(h, a0, a1, d0, d1, W2aT[l], W2bT[l], b2r[l], W1aT[l + 1])
    else:
      out = _final(h, a0, a1, d0, d1, W2aT[4], W2bT[4], b2r[4],
                   L1W.T, L1b.reshape(1, F), L2W.T, L2b.reshape(1, D))
  return out


# speedup vs baseline: 7.9925x; 1.8205x over previous
import functools

import jax
import jax.numpy as jnp
from jax import lax
from jax.experimental import pallas as pl
from jax.experimental.pallas import tpu as pltpu
from jax.experimental.pallas import tpu_sc as plsc

N = 10000
E = 320000
F = 128
D = 32
NC, NS = 2, 16
RW = 128
ROWS = E // RW
ROWS_PER_SC = ROWS // NC
NPS = N // NS
NB = 400
GRID_N = N // NB
EB = 6400
GRID_E = E // EB

_HI = lax.Precision.HIGHEST


def _dot(a, b):
  return jnp.dot(a, b, preferred_element_type=jnp.float32, precision=_HI)


G = 6
NBLK = 13
RPT = G * NBLK
GE = G * RW


def _sc_edge_body(with_deg, *refs):
  if with_deg:
    (p_hbm, es_hbm, ed_hbm, er_hbm, w1b_hbm, src_hbm, dst_hbm,
     acc0_hbm, acc1_hbm, deg0_hbm, deg1_hbm,
     idx2, dst2, dsts, rows2, es2, ed2, er2, w1b_v, ones_v, vz_v,
     acc_sh, deg_sh,
     sem_ld0, sem_ld1, sem_g0, sem_g1, sem_s0, sem_s1) = refs
  else:
    (p_hbm, es_hbm, ed_hbm, er_hbm, w1b_hbm, src_hbm, dst_hbm,
     acc0_hbm, acc1_hbm,
     idx2, dst2, dsts, rows2, es2, ed2, er2, w1b_v,
     acc_sh,
     sem_ld0, sem_ld1, sem_g0, sem_g1, sem_s0, sem_s1) = refs
  sem_ld = (sem_ld0, sem_ld1)
  sem_g = (sem_g0, sem_g1)
  sem_s = (sem_s0, sem_s1)

  cid = lax.axis_index("c")
  sid = lax.axis_index("s")
  wid = sid * NC + cid
  zv = jnp.zeros((16,), jnp.float32)

  def zero_rows(e, c):
    rows2[0, e, pl.ds(0, 16)] = zv
    rows2[0, e, pl.ds(16, 16)] = zv
    return c
  lax.fori_loop(0, RW, zero_rows, 0)
  rows_v = rows2.at[0, pl.ds(0, RW)]
  zbase = sid * 640

  @pl.when(sid < NS - 1)
  def _():
    for off in (0, 128, 256, 384, 512):
      pltpu.sync_copy(rows_v, acc_sh.at[pl.ds(zbase + off, 128)])

  @pl.when(sid == NS - 1)
  def _():
    for off in (0, 128, 256):
      pltpu.sync_copy(rows_v, acc_sh.at[pl.ds(zbase + off, 128)])
    pltpu.sync_copy(rows2.at[0, pl.ds(0, 16)],
                    acc_sh.at[pl.ds(zbase + 384, 16)])

  if with_deg:
    def zero_vz(i, c):
      vz_v[pl.ds(i * 16, 16)] = zv
      return c
    lax.fori_loop(0, 40, zero_vz, 0)
    dzb = sid * 640

    @pl.when(sid < NS - 1)
    def _():
      pltpu.sync_copy(vz_v, deg_sh.at[pl.ds(dzb, 640)])

    @pl.when(sid == NS - 1)
    def _():
      pltpu.sync_copy(vz_v.at[pl.ds(0, 400)], deg_sh.at[pl.ds(dzb, 400)])

    ov = jnp.ones((16,), jnp.float32)
    def fill_ones(i, c):
      ones_v[pl.ds(i * 16, 16)] = ov
      return c
    lax.fori_loop(0, 8, fill_ones, 0)

  pltpu.sync_copy(w1b_hbm, w1b_v)
  cvecs = [[w1b_v[k, pl.ds(h * 16, 16)] for h in range(2)] for k in range(3)]

  plsc.subcore_barrier()

  base_row = wid * RPT

  def issue_loads(b, buf):
    r0 = base_row + b * G
    return [
        pltpu.async_copy(src_hbm.at[pl.ds(r0, G)], idx2.at[buf], sem_ld[buf]),
        pltpu.async_copy(dst_hbm.at[pl.ds(r0, G)], dst2.at[buf], sem_ld[buf]),
        pltpu.async_copy(es_hbm.at[pl.ds(r0 * RW, GE)], es2.at[buf],
                         sem_ld[buf]),
        pltpu.async_copy(ed_hbm.at[pl.ds(r0 * RW, GE)], ed2.at[buf],
                         sem_ld[buf]),
        pltpu.async_copy(er_hbm.at[pl.ds(r0 * RW, GE)], er2.at[buf],
                         sem_ld[buf]),
    ]

  def issue_gathers(buf):
    return [
        pltpu.async_copy(p_hbm.at[idx2.at[buf, j]],
                         rows2.at[buf, pl.ds(j * RW, RW)], sem_g[buf])
        for j in range(G)
    ]

  def compute_block(buf, n_edges):
    def comp(e, cc):
      eidx = jnp.full((16,), e, dtype=jnp.int32)
      se = plsc.load_gather(es2.at[buf], [eidx])
      sd = plsc.load_gather(ed2.at[buf], [eidx])
      sr = plsc.load_gather(er2.at[buf], [eidx])
      for h, off in ((0, 0), (1, 16)):
        wqv = se * cvecs[0][h] + sd * cvecs[1][h] + sr * cvecs[2][h]
        a = rows2[buf, e, pl.ds(off, 16)] + wqv
        rows2[buf, e, pl.ds(off, 16)] = jnp.maximum(a, a * 0.01)
      return cc
    lax.fori_loop(0, n_edges, comp, 0)

  def issue_scatters(buf):
    def cp(i, c):
      j = i // 8
      k = (i % 8) * 16
      dsts[buf, j, pl.ds(k, 16)] = dst2[buf, j, pl.ds(k, 16)]
      return c
    lax.fori_loop(0, G * 8, cp, 0)
    ds_ = [
        pltpu.async_copy(rows2.at[buf, pl.ds(j * RW, RW)],
                         acc_sh.at[dsts.at[buf, j]], sem_s[buf], add=True)
        for j in range(G)
    ]
    if with_deg:
      ds_ += [
          pltpu.async_copy(ones_v, deg_sh.at[dsts.at[buf, j]], sem_s[buf],
                           add=True)
          for j in range(G)
      ]
    return ds_

  ld = [None, None]
  g = [None, None]
  s = [None, None]
  ld[0] = issue_loads(0, 0)
  for d in ld[0]:
    d.wait()
  g[0] = issue_gathers(0)
  ld[1] = issue_loads(1, 1)
  for b in range(NBLK):
    buf = b % 2
    obuf = 1 - buf
    for d in g[buf]:
      d.wait()
    if b + 1 < NBLK:
      for d in ld[obuf]:
        d.wait()
      if b >= 1:
        for d in s[obuf]:
          d.wait()
      g[obuf] = issue_gathers(obuf)
    compute_block(buf, GE)
    s[buf] = issue_scatters(buf)
    if b + 2 < NBLK:
      ld[buf] = issue_loads(b + 2, buf)
  for d in s[0] + s[1]:
    d.wait()

  @pl.when(wid < ROWS - 32 * RPT)
  def _():
    trow = 32 * RPT + wid
    pltpu.async_copy(src_hbm.at[pl.ds(trow, 1)], idx2.at[0, pl.ds(0, 1)],
                     sem_ld[0]).wait()
    pltpu.async_copy(dst_hbm.at[pl.ds(trow, 1)], dst2.at[0, pl.ds(0, 1)],
                     sem_ld[0]).wait()
    pltpu.async_copy(es_hbm.at[pl.ds(trow * RW, RW)],
                     es2.at[0, pl.ds(0, RW)], sem_ld[0]).wait()
    pltpu.async_copy(ed_hbm.at[pl.ds(trow * RW, RW)],
                     ed2.at[0, pl.ds(0, RW)], sem_ld[0]).wait()
    pltpu.async_copy(er_hbm.at[pl.ds(trow * RW, RW)],
                     er2.at[0, pl.ds(0, RW)], sem_ld[0]).wait()
    pltpu.async_copy(p_hbm.at[idx2.at[0, 0]], rows2.at[0, pl.ds(0, RW)],
                     sem_g[0]).wait()
    compute_block(0, RW)
    pltpu.async_copy(rows2.at[0, pl.ds(0, RW)], acc_sh.at[dst2.at[0, 0]],
                     sem_s[0], add=True).wait()
    if with_deg:
      pltpu.async_copy(ones_v, deg_sh.at[dst2.at[0, 0]], sem_s[0],
                       add=True).wait()

  plsc.subcore_barrier()

  for cc, acc_hbm in ((0, acc0_hbm), (1, acc1_hbm)):
    @pl.when((sid < NS - 1) & (cid == cc))
    def _(acc_hbm=acc_hbm):
      pltpu.sync_copy(acc_sh.at[pl.ds(zbase, 640)],
                      acc_hbm.at[pl.ds(zbase, 640)])

    @pl.when((sid == NS - 1) & (cid == cc))
    def _(acc_hbm=acc_hbm):
      pltpu.sync_copy(acc_sh.at[pl.ds(zbase, 400)],
                      acc_hbm.at[pl.ds(zbase, 400)])

  if with_deg:
    dzb = sid * 640
    ln_tail = 400

    @pl.when((sid < NS - 1) & (cid == 0))
    def _():
      pltpu.sync_copy(deg_sh.at[pl.ds(dzb, 640)], deg0_hbm.at[pl.ds(dzb, 640)])

    @pl.when((sid == NS - 1) & (cid == 0))
    def _():
      pltpu.sync_copy(deg_sh.at[pl.ds(dzb, ln_tail)],
                      deg0_hbm.at[pl.ds(dzb, ln_tail)])

    @pl.when((sid < NS - 1) & (cid == 1))
    def _():
      pltpu.sync_copy(deg_sh.at[pl.ds(dzb, 640)], deg1_hbm.at[pl.ds(dzb, 640)])

    @pl.when((sid == NS - 1) & (cid == 1))
    def _():
      pltpu.sync_copy(deg_sh.at[pl.ds(dzb, ln_tail)],
                      deg1_hbm.at[pl.ds(dzb, ln_tail)])


_SC_MESH = plsc.VectorSubcoreMesh(
    core_axis_name="c", subcore_axis_name="s", num_cores=NC, num_subcores=NS)

_SC_PARAMS = pltpu.CompilerParams(use_tc_tiling_on_sc=False,
                                  needs_layout_passes=False)

_SC_SCRATCH_COMMON = [
    pltpu.VMEM((2, G, RW), jnp.int32),
    pltpu.VMEM((2, G, RW), jnp.int32),
    pltpu.VMEM((2, G, RW), jnp.int32),
    pltpu.VMEM((2, GE, D), jnp.float32),
    pltpu.VMEM((2, GE), jnp.float32),
    pltpu.VMEM((2, GE), jnp.float32),
    pltpu.VMEM((2, GE), jnp.float32),
    pltpu.VMEM((3, D), jnp.float32),
]

_SC_SEMS = [pltpu.SemaphoreType.DMA] * 6

_sc_edge_deg = pl.kernel(
    functools.partial(_sc_edge_body, True),
    out_type=(jax.ShapeDtypeStruct((N, D), jnp.float32),
              jax.ShapeDtypeStruct((N, D), jnp.float32),
              jax.ShapeDtypeStruct((N,), jnp.float32),
              jax.ShapeDtypeStruct((N,), jnp.float32)),
    mesh=_SC_MESH,
    scratch_types=_SC_SCRATCH_COMMON + [
        pltpu.VMEM((RW,), jnp.float32),
        pltpu.VMEM((640,), jnp.float32),
        pltpu.VMEM_SHARED((N, D), jnp.float32),
        pltpu.VMEM_SHARED((N,), jnp.float32),
    ] + _SC_SEMS,
    compiler_params=_SC_PARAMS,
    name="sc_edge_deg",
)

_sc_edge = pl.kernel(
    functools.partial(_sc_edge_body, False),
    out_type=(jax.ShapeDtypeStruct((N, D), jnp.float32),
              jax.ShapeDtypeStruct((N, D), jnp.float32)),
    mesh=_SC_MESH,
    scratch_types=_SC_SCRATCH_COMMON + [
        pltpu.VMEM_SHARED((N, D), jnp.float32),
    ] + _SC_SEMS,
    compiler_params=_SC_PARAMS,
    name="sc_edge",
)


def _prep_body(gate_ref, emb_ref, w1aT_ref, h0_ref, p0_ref):
  g = gate_ref[...]
  iota = lax.broadcasted_iota(jnp.int32, (NB, F), 1)
  oh = jnp.where(g == iota, 1.0, 0.0).astype(jnp.float32)
  h0 = _dot(oh, emb_ref[...])
  h0_ref[...] = h0
  p0_ref[...] = _dot(h0, w1aT_ref[...])


_prep = pl.pallas_call(
    _prep_body,
    grid=(GRID_N,),
    in_specs=[
        pl.BlockSpec((NB, 1), lambda i: (i, 0)),
        pl.BlockSpec((F, F), lambda i: (0, 0)),
        pl.BlockSpec((F, D), lambda i: (0, 0)),
    ],
    out_specs=(pl.BlockSpec((NB, F), lambda i: (i, 0)),
               pl.BlockSpec((NB, D), lambda i: (i, 0))),
    out_shape=(jax.ShapeDtypeStruct((N, F), jnp.float32),
               jax.ShapeDtypeStruct((N, D), jnp.float32)),
)


def _combine(a0, a1, d0, d1):
  deg = jnp.maximum(d0 + d1, 1.0)
  return (a0 + a1) / deg


def _layer_body(h_ref, a0_ref, a1_ref, d0_ref, d1_ref, w2aT_ref, w2bT_ref,
                b2_ref, w1aT_ref, hout_ref, pout_ref):
  hN = _combine(a0_ref[...], a1_ref[...], d0_ref[...], d1_ref[...])
  z = _dot(h_ref[...], w2aT_ref[...]) + _dot(hN, w2bT_ref[...]) + b2_ref[...]
  hn = jnp.maximum(z, 0.0)
  hout_ref[...] = hn
  pout_ref[...] = _dot(hn, w1aT_ref[...])


_layer = pl.pallas_call(
    _layer_body,
    grid=(GRID_N,),
    in_specs=[
        pl.BlockSpec((NB, F), lambda i: (i, 0)),
        pl.BlockSpec((NB, D), lambda i: (i, 0)),
        pl.BlockSpec((NB, D), lambda i: (i, 0)),
        pl.BlockSpec((NB, 1), lambda i: (i, 0)),
        pl.BlockSpec((NB, 1), lambda i: (i, 0)),
        pl.BlockSpec((F, F), lambda i: (0, 0)),
        pl.BlockSpec((D, F), lambda i: (0, 0)),
        pl.BlockSpec((1, F), lambda i: (0, 0)),
        pl.BlockSpec((F, D), lambda i: (0, 0)),
    ],
    out_specs=(pl.BlockSpec((NB, F), lambda i: (i, 0)),
               pl.BlockSpec((NB, D), lambda i: (i, 0))),
    out_shape=(jax.ShapeDtypeStruct((N, F), jnp.float32),
               jax.ShapeDtypeStruct((N, D), jnp.float32)),
)


def _final_body(h_ref, a0_ref, a1_ref, d0_ref, d1_ref, w2aT_ref, w2bT_ref,
                b2_ref, l1wT_ref, l1b_ref, l2wT_ref, l2b_ref, out_ref):
  hN = _combine(a0_ref[...], a1_ref[...], d0_ref[...], d1_ref[...])
  z = _dot(h_ref[...], w2aT_ref[...]) + _dot(hN, w2bT_ref[...]) + b2_ref[...]
  hn = jnp.maximum(z, 0.0)
  h1 = jnp.maximum(_dot(hn, l1wT_ref[...]) + l1b_ref[...], 0.0)
  out_ref[...] = _dot(h1, l2wT_ref[...]) + l2b_ref[...]


_final = pl.pallas_call(
    _final_body,
    grid=(GRID_N,),
    in_specs=[
        pl.BlockSpec((NB, F), lambda i: (i, 0)),
        pl.BlockSpec((NB, D), lambda i: (i, 0)),
        pl.BlockSpec((NB, D), lambda i: (i, 0)),
        pl.BlockSpec((NB, 1), lambda i: (i, 0)),
        pl.BlockSpec((NB, 1), lambda i: (i, 0)),
        pl.BlockSpec((F, F), lambda i: (0, 0)),
        pl.BlockSpec((D, F), lambda i: (0, 0)),
        pl.BlockSpec((1, F), lambda i: (0, 0)),
        pl.BlockSpec((F, F), lambda i: (0, 0)),
        pl.BlockSpec((1, F), lambda i: (0, 0)),
        pl.BlockSpec((F, D), lambda i: (0, 0)),
        pl.BlockSpec((1, D), lambda i: (0, 0)),
    ],
    out_specs=pl.BlockSpec((NB, D), lambda i: (i, 0)),
    out_shape=jax.ShapeDtypeStruct((N, D), jnp.float32),
)


def kernel(gate_type, edge_index, edge_src_idx, edge_dst_idx, edge_reversed,
           emb, W1_0, W2_0, b2_0, W1_1, W2_1, b2_1, W1_2, W2_2, b2_2,
           W1_3, W2_3, b2_3, W1_4, W2_4, b2_4, L1W, L1b, L2W, L2b):
  gate2d = gate_type.astype(jnp.int32).reshape(N, 1)
  src2d = edge_index[0].astype(jnp.int32).reshape(ROWS, RW)
  dst2d = edge_index[1].astype(jnp.int32).reshape(ROWS, RW)

  convs = [(W1_0, W2_0, b2_0), (W1_1, W2_1, b2_1), (W1_2, W2_2, b2_2),
           (W1_3, W2_3, b2_3), (W1_4, W2_4, b2_4)]
  W1aT, W1bT, W2aT, W2bT, b2r = [], [], [], [], []
  for W1, W2, b2 in convs:
    W1aT.append(W1[:, :F].T)
    W1bT.append(W1[:, F:F + 3].T)
    W2aT.append(W2[:, :F].T)
    W2bT.append(W2[:, F:].T)
    b2r.append(b2.reshape(1, F))

  h, p = _prep(gate2d, emb, W1aT[0])

  out = None
  d0 = d1 = None
  for l in range(5):
    if l == 0:
      a0, a1, deg0, deg1 = _sc_edge_deg(
          p, edge_src_idx, edge_dst_idx, edge_reversed, W1bT[l], src2d, dst2d)
      d0 = deg0.reshape(N, 1)
      d1 = deg1.reshape(N, 1)
    else:
      a0, a1 = _sc_edge(
          p, edge_src_idx, edge_dst_idx, edge_reversed, W1bT[l], src2d, dst2d)
    if l < 4:
      h, p = _layer(h, a0, a1, d0, d1, W2aT[l], W2bT[l], b2r[l], W1aT[l + 1])
    else:
      out = _final(h, a0, a1, d0, d1, W2aT[4], W2bT[4], b2r[4],
                   L1W.T, L1b.reshape(1, F), L2W.T, L2b.reshape(1, D))
  return out


# speedup vs baseline: 12.5692x; 1.5726x over previous
import functools

import jax
import jax.numpy as jnp
from jax import lax
from jax.experimental import pallas as pl
from jax.experimental.pallas import tpu as pltpu
from jax.experimental.pallas import tpu_sc as plsc

N = 10000
E = 320000
F = 128
D = 32
NC, NS = 2, 16
RW = 128
ROWS = E // RW
ROWS_PER_SC = ROWS // NC
NPS = N // NS
NB = 400
GRID_N = N // NB
EB = 6400
GRID_E = E // EB

_HI = lax.Precision.HIGHEST


def _dot(a, b):
  return jnp.dot(a, b, preferred_element_type=jnp.float32, precision=_HI)


G = 6
NBLK = 13
RPT = G * NBLK
GE = G * RW


def _sc_edge_body(with_deg, *refs):
  if with_deg:
    (p_hbm, es_hbm, ed_hbm, er_hbm, w1b_hbm, src_hbm, dst_hbm,
     acc0_hbm, acc1_hbm, deg0_hbm, deg1_hbm,
     idx2, dst2, dsts, rows2, es2, ed2, er2, w1b_v, ones_v, vz_v,
     acc_sh, deg_sh,
     sem_ld0, sem_ld1, sem_g0, sem_g1, sem_s0, sem_s1) = refs
  else:
    (p_hbm, es_hbm, ed_hbm, er_hbm, w1b_hbm, src_hbm, dst_hbm,
     acc0_hbm, acc1_hbm,
     idx2, dst2, dsts, rows2, es2, ed2, er2, w1b_v,
     acc_sh,
     sem_ld0, sem_ld1, sem_g0, sem_g1, sem_s0, sem_s1) = refs
  sem_ld = (sem_ld0, sem_ld1)
  sem_g = (sem_g0, sem_g1)
  sem_s = (sem_s0, sem_s1)

  cid = lax.axis_index("c")
  sid = lax.axis_index("s")
  wid = sid * NC + cid
  zv = jnp.zeros((16,), jnp.float32)

  def zero_rows(e, c):
    rows2[0, e, pl.ds(0, 16)] = zv
    rows2[0, e, pl.ds(16, 16)] = zv
    return c
  lax.fori_loop(0, RW, zero_rows, 0)
  rows_v = rows2.at[0, pl.ds(0, RW)]
  zbase = sid * 640

  @pl.when(sid < NS - 1)
  def _():
    for off in (0, 128, 256, 384, 512):
      pltpu.sync_copy(rows_v, acc_sh.at[pl.ds(zbase + off, 128)])

  @pl.when(sid == NS - 1)
  def _():
    for off in (0, 128, 256):
      pltpu.sync_copy(rows_v, acc_sh.at[pl.ds(zbase + off, 128)])
    pltpu.sync_copy(rows2.at[0, pl.ds(0, 16)],
                    acc_sh.at[pl.ds(zbase + 384, 16)])

  if with_deg:
    def zero_vz(i, c):
      vz_v[pl.ds(i * 16, 16)] = zv
      return c
    lax.fori_loop(0, 40, zero_vz, 0)
    dzb = sid * 640

    @pl.when(sid < NS - 1)
    def _():
      pltpu.sync_copy(vz_v, deg_sh.at[pl.ds(dzb, 640)])

    @pl.when(sid == NS - 1)
    def _():
      pltpu.sync_copy(vz_v.at[pl.ds(0, 400)], deg_sh.at[pl.ds(dzb, 400)])

    ov = jnp.ones((16,), jnp.float32)
    def fill_ones(i, c):
      ones_v[pl.ds(i * 16, 16)] = ov
      return c
    lax.fori_loop(0, 8, fill_ones, 0)

  pltpu.sync_copy(w1b_hbm, w1b_v)
  cvecs = [[w1b_v[k, pl.ds(h * 16, 16)] for h in range(2)] for k in range(3)]

  plsc.subcore_barrier()

  base_row = wid * RPT

  def issue_loads(b, buf):
    r0 = base_row + b * G
    return [
        pltpu.async_copy(src_hbm.at[pl.ds(r0, G)], idx2.at[buf], sem_ld[buf]),
        pltpu.async_copy(dst_hbm.at[pl.ds(r0, G)], dst2.at[buf], sem_ld[buf]),
        pltpu.async_copy(es_hbm.at[pl.ds(r0 * RW, GE)], es2.at[buf],
                         sem_ld[buf]),
        pltpu.async_copy(ed_hbm.at[pl.ds(r0 * RW, GE)], ed2.at[buf],
                         sem_ld[buf]),
        pltpu.async_copy(er_hbm.at[pl.ds(r0 * RW, GE)], er2.at[buf],
                         sem_ld[buf]),
    ]

  def issue_gathers(buf):
    return [
        pltpu.async_copy(p_hbm.at[idx2.at[buf, j]],
                         rows2.at[buf, pl.ds(j * RW, RW)], sem_g[buf])
        for j in range(G)
    ]

  def compute_block(buf, n_edges):
    @plsc.parallel_loop(0, n_edges, unroll=4)
    def _(e):
      eidx = jnp.full((16,), e, dtype=jnp.int32)
      se = plsc.load_gather(es2.at[buf], [eidx])
      sd = plsc.load_gather(ed2.at[buf], [eidx])
      sr = plsc.load_gather(er2.at[buf], [eidx])
      for h, off in ((0, 0), (1, 16)):
        wqv = se * cvecs[0][h] + sd * cvecs[1][h] + sr * cvecs[2][h]
        a = rows2[buf, e, pl.ds(off, 16)] + wqv
        rows2[buf, e, pl.ds(off, 16)] = jnp.maximum(a, a * 0.01)

  def issue_scatters(buf):
    def cp(i, c):
      j = i // 8
      k = (i % 8) * 16
      dsts[buf, j, pl.ds(k, 16)] = dst2[buf, j, pl.ds(k, 16)]
      return c
    lax.fori_loop(0, G * 8, cp, 0)
    ds_ = [
        pltpu.async_copy(rows2.at[buf, pl.ds(j * RW, RW)],
                         acc_sh.at[dsts.at[buf, j]], sem_s[buf], add=True)
        for j in range(G)
    ]
    if with_deg:
      ds_ += [
          pltpu.async_copy(ones_v, deg_sh.at[dsts.at[buf, j]], sem_s[buf],
                           add=True)
          for j in range(G)
      ]
    return ds_

  ld = [None, None]
  g = [None, None]
  s = [None, None]
  ld[0] = issue_loads(0, 0)
  for d in ld[0]:
    d.wait()
  g[0] = issue_gathers(0)
  ld[1] = issue_loads(1, 1)
  for b in range(NBLK):
    buf = b % 2
    obuf = 1 - buf
    for d in g[buf]:
      d.wait()
    if b + 1 < NBLK:
      for d in ld[obuf]:
        d.wait()
      if b >= 1:
        for d in s[obuf]:
          d.wait()
      g[obuf] = issue_gathers(obuf)
    compute_block(buf, GE)
    s[buf] = issue_scatters(buf)
    if b + 2 < NBLK:
      ld[buf] = issue_loads(b + 2, buf)
  for d in s[0] + s[1]:
    d.wait()

  @pl.when(wid < ROWS - 32 * RPT)
  def _():
    trow = 32 * RPT + wid
    pltpu.async_copy(src_hbm.at[pl.ds(trow, 1)], idx2.at[0, pl.ds(0, 1)],
                     sem_ld[0]).wait()
    pltpu.async_copy(dst_hbm.at[pl.ds(trow, 1)], dst2.at[0, pl.ds(0, 1)],
                     sem_ld[0]).wait()
    pltpu.async_copy(es_hbm.at[pl.ds(trow * RW, RW)],
                     es2.at[0, pl.ds(0, RW)], sem_ld[0]).wait()
    pltpu.async_copy(ed_hbm.at[pl.ds(trow * RW, RW)],
                     ed2.at[0, pl.ds(0, RW)], sem_ld[0]).wait()
    pltpu.async_copy(er_hbm.at[pl.ds(trow * RW, RW)],
                     er2.at[0, pl.ds(0, RW)], sem_ld[0]).wait()
    pltpu.async_copy(p_hbm.at[idx2.at[0, 0]], rows2.at[0, pl.ds(0, RW)],
                     sem_g[0]).wait()
    compute_block(0, RW)
    pltpu.async_copy(rows2.at[0, pl.ds(0, RW)], acc_sh.at[dst2.at[0, 0]],
                     sem_s[0], add=True).wait()
    if with_deg:
      pltpu.async_copy(ones_v, deg_sh.at[dst2.at[0, 0]], sem_s[0],
                       add=True).wait()

  plsc.subcore_barrier()

  for cc, acc_hbm in ((0, acc0_hbm), (1, acc1_hbm)):
    @pl.when((sid < NS - 1) & (cid == cc))
    def _(acc_hbm=acc_hbm):
      pltpu.sync_copy(acc_sh.at[pl.ds(zbase, 640)],
                      acc_hbm.at[pl.ds(zbase, 640)])

    @pl.when((sid == NS - 1) & (cid == cc))
    def _(acc_hbm=acc_hbm):
      pltpu.sync_copy(acc_sh.at[pl.ds(zbase, 400)],
                      acc_hbm.at[pl.ds(zbase, 400)])

  if with_deg:
    dzb = sid * 640
    ln_tail = 400

    @pl.when((sid < NS - 1) & (cid == 0))
    def _():
      pltpu.sync_copy(deg_sh.at[pl.ds(dzb, 640)], deg0_hbm.at[pl.ds(dzb, 640)])

    @pl.when((sid == NS - 1) & (cid == 0))
    def _():
      pltpu.sync_copy(deg_sh.at[pl.ds(dzb, ln_tail)],
                      deg0_hbm.at[pl.ds(dzb, ln_tail)])

    @pl.when((sid < NS - 1) & (cid == 1))
    def _():
      pltpu.sync_copy(deg_sh.at[pl.ds(dzb, 640)], deg1_hbm.at[pl.ds(dzb, 640)])

    @pl.when((sid == NS - 1) & (cid == 1))
    def _():
      pltpu.sync_copy(deg_sh.at[pl.ds(dzb, ln_tail)],
                      deg1_hbm.at[pl.ds(dzb, ln_tail)])


_SC_MESH = plsc.VectorSubcoreMesh(
    core_axis_name="c", subcore_axis_name="s", num_cores=NC, num_subcores=NS)

_SC_PARAMS = pltpu.CompilerParams(use_tc_tiling_on_sc=False,
                                  needs_layout_passes=False)

_SC_SCRATCH_COMMON = [
    pltpu.VMEM((2, G, RW), jnp.int32),
    pltpu.VMEM((2, G, RW), jnp.int32),
    pltpu.VMEM((2, G, RW), jnp.int32),
    pltpu.VMEM((2, GE, D), jnp.float32),
    pltpu.VMEM((2, GE), jnp.float32),
    pltpu.VMEM((2, GE), jnp.float32),
    pltpu.VMEM((2, GE), jnp.float32),
    pltpu.VMEM((3, D), jnp.float32),
]

_SC_SEMS = [pltpu.SemaphoreType.DMA] * 6

_sc_edge_deg = pl.kernel(
    functools.partial(_sc_edge_body, True),
    out_type=(jax.ShapeDtypeStruct((N, D), jnp.float32),
              jax.ShapeDtypeStruct((N, D), jnp.float32),
              jax.ShapeDtypeStruct((N,), jnp.float32),
              jax.ShapeDtypeStruct((N,), jnp.float32)),
    mesh=_SC_MESH,
    scratch_types=_SC_SCRATCH_COMMON + [
        pltpu.VMEM((RW,), jnp.float32),
        pltpu.VMEM((640,), jnp.float32),
        pltpu.VMEM_SHARED((N, D), jnp.float32),
        pltpu.VMEM_SHARED((N,), jnp.float32),
    ] + _SC_SEMS,
    compiler_params=_SC_PARAMS,
    name="sc_edge_deg",
)

_sc_edge = pl.kernel(
    functools.partial(_sc_edge_body, False),
    out_type=(jax.ShapeDtypeStruct((N, D), jnp.float32),
              jax.ShapeDtypeStruct((N, D), jnp.float32)),
    mesh=_SC_MESH,
    scratch_types=_SC_SCRATCH_COMMON + [
        pltpu.VMEM_SHARED((N, D), jnp.float32),
    ] + _SC_SEMS,
    compiler_params=_SC_PARAMS,
    name="sc_edge",
)


def _prep_body(gate_ref, emb_ref, w1aT_ref, h0_ref, p0_ref):
  g = gate_ref[...]
  iota = lax.broadcasted_iota(jnp.int32, (NB, F), 1)
  oh = jnp.where(g == iota, 1.0, 0.0).astype(jnp.float32)
  h0 = _dot(oh, emb_ref[...])
  h0_ref[...] = h0
  p0_ref[...] = _dot(h0, w1aT_ref[...])


_prep = pl.pallas_call(
    _prep_body,
    grid=(GRID_N,),
    in_specs=[
        pl.BlockSpec((NB, 1), lambda i: (i, 0)),
        pl.BlockSpec((F, F), lambda i: (0, 0)),
        pl.BlockSpec((F, D), lambda i: (0, 0)),
    ],
    out_specs=(pl.BlockSpec((NB, F), lambda i: (i, 0)),
               pl.BlockSpec((NB, D), lambda i: (i, 0))),
    out_shape=(jax.ShapeDtypeStruct((N, F), jnp.float32),
               jax.ShapeDtypeStruct((N, D), jnp.float32)),
)


def _combine(a0, a1, d0, d1):
  deg = jnp.maximum(d0 + d1, 1.0)
  return (a0 + a1) / deg


def _layer_body(h_ref, a0_ref, a1_ref, d0_ref, d1_ref, w2aT_ref, w2bT_ref,
                b2_ref, w1aT_ref, hout_ref, pout_ref):
  hN = _combine(a0_ref[...], a1_ref[...], d0_ref[...], d1_ref[...])
  z = _dot(h_ref[...], w2aT_ref[...]) + _dot(hN, w2bT_ref[...]) + b2_ref[...]
  hn = jnp.maximum(z, 0.0)
  hout_ref[...] = hn
  pout_ref[...] = _dot(hn, w1aT_ref[...])


_layer = pl.pallas_call(
    _layer_body,
    grid=(GRID_N,),
    in_specs=[
        pl.BlockSpec((NB, F), lambda i: (i, 0)),
        pl.BlockSpec((NB, D), lambda i: (i, 0)),
        pl.BlockSpec((NB, D), lambda i: (i, 0)),
        pl.BlockSpec((NB, 1), lambda i: (i, 0)),
        pl.BlockSpec((NB, 1), lambda i: (i, 0)),
        pl.BlockSpec((F, F), lambda i: (0, 0)),
        pl.BlockSpec((D, F), lambda i: (0, 0)),
        pl.BlockSpec((1, F), lambda i: (0, 0)),
        pl.BlockSpec((F, D), lambda i: (0, 0)),
    ],
    out_specs=(pl.BlockSpec((NB, F), lambda i: (i, 0)),
               pl.BlockSpec((NB, D), lambda i: (i, 0))),
    out_shape=(jax.ShapeDtypeStruct((N, F), jnp.float32),
               jax.ShapeDtypeStruct((N, D), jnp.float32)),
)


def _final_body(h_ref, a0_ref, a1_ref, d0_ref, d1_ref, w2aT_ref, w2bT_ref,
                b2_ref, l1wT_ref, l1b_ref, l2wT_ref, l2b_ref, out_ref):
  hN = _combine(a0_ref[...], a1_ref[...], d0_ref[...], d1_ref[...])
  z = _dot(h_ref[...], w2aT_ref[...]) + _dot(hN, w2bT_ref[...]) + b2_ref[...]
  hn = jnp.maximum(z, 0.0)
  h1 = jnp.maximum(_dot(hn, l1wT_ref[...]) + l1b_ref[...], 0.0)
  out_ref[...] = _dot(h1, l2wT_ref[...]) + l2b_ref[...]


_final = pl.pallas_call(
    _final_body,
    grid=(GRID_N,),
    in_specs=[
        pl.BlockSpec((NB, F), lambda i: (i, 0)),
        pl.BlockSpec((NB, D), lambda i: (i, 0)),
        pl.BlockSpec((NB, D), lambda i: (i, 0)),
        pl.BlockSpec((NB, 1), lambda i: (i, 0)),
        pl.BlockSpec((NB, 1), lambda i: (i, 0)),
        pl.BlockSpec((F, F), lambda i: (0, 0)),
        pl.BlockSpec((D, F), lambda i: (0, 0)),
        pl.BlockSpec((1, F), lambda i: (0, 0)),
        pl.BlockSpec((F, F), lambda i: (0, 0)),
        pl.BlockSpec((1, F), lambda i: (0, 0)),
        pl.BlockSpec((F, D), lambda i: (0, 0)),
        pl.BlockSpec((1, D), lambda i: (0, 0)),
    ],
    out_specs=pl.BlockSpec((NB, D), lambda i: (i, 0)),
    out_shape=jax.ShapeDtypeStruct((N, D), jnp.float32),
)


def kernel(gate_type, edge_index, edge_src_idx, edge_dst_idx, edge_reversed,
           emb, W1_0, W2_0, b2_0, W1_1, W2_1, b2_1, W1_2, W2_2, b2_2,
           W1_3, W2_3, b2_3, W1_4, W2_4, b2_4, L1W, L1b, L2W, L2b):
  gate2d = gate_type.astype(jnp.int32).reshape(N, 1)
  src2d = edge_index[0].astype(jnp.int32).reshape(ROWS, RW)
  dst2d = edge_index[1].astype(jnp.int32).reshape(ROWS, RW)

  convs = [(W1_0, W2_0, b2_0), (W1_1, W2_1, b2_1), (W1_2, W2_2, b2_2),
           (W1_3, W2_3, b2_3), (W1_4, W2_4, b2_4)]
  W1aT, W1bT, W2aT, W2bT, b2r = [], [], [], [], []
  for W1, W2, b2 in convs:
    W1aT.append(W1[:, :F].T)
    W1bT.append(W1[:, F:F + 3].T)
    W2aT.append(W2[:, :F].T)
    W2bT.append(W2[:, F:].T)
    b2r.append(b2.reshape(1, F))

  h, p = _prep(gate2d, emb, W1aT[0])

  out = None
  d0 = d1 = None
  for l in range(5):
    if l == 0:
      a0, a1, deg0, deg1 = _sc_edge_deg(
          p, edge_src_idx, edge_dst_idx, edge_reversed, W1bT[l], src2d, dst2d)
      d0 = deg0.reshape(N, 1)
      d1 = deg1.reshape(N, 1)
    else:
      a0, a1 = _sc_edge(
          p, edge_src_idx, edge_dst_idx, edge_reversed, W1bT[l], src2d, dst2d)
    if l < 4:
      h, p = _layer(h, a0, a1, d0, d1, W2aT[l], W2bT[l], b2r[l], W1aT[l + 1])
    else:
      out = _final(h, a0, a1, d0, d1, W2aT[4], W2bT[4], b2r[4],
                   L1W.T, L1b.reshape(1, F), L2W.T, L2b.reshape(1, D))
  return out


# speedup vs baseline: 13.2514x; 1.0543x over previous
import functools

import jax
import jax.numpy as jnp
from jax import lax
from jax.experimental import pallas as pl
from jax.experimental.pallas import tpu as pltpu
from jax.experimental.pallas import tpu_sc as plsc

N = 10000
E = 320000
F = 128
D = 32
NC, NS = 2, 16
RW = 128
ROWS = E // RW
ROWS_PER_SC = ROWS // NC
NPS = N // NS
NB = 2000
GRID_N = N // NB
EB = 6400
GRID_E = E // EB

_HI = lax.Precision.HIGHEST


def _dot(a, b):
  return jnp.dot(a, b, preferred_element_type=jnp.float32, precision=_HI)


G = 6
NBLK = 13
RPT = G * NBLK
GE = G * RW


def _sc_edge_body(with_deg, *refs):
  if with_deg:
    (p_hbm, es_hbm, ed_hbm, er_hbm, w1b_hbm, src_hbm, dst_hbm,
     acc0_hbm, acc1_hbm, deg0_hbm, deg1_hbm,
     idx2, dst2, rows2, es2, ed2, er2, w1b_v, ones_v, vz_v,
     acc_sh, deg_sh,
     sem_ld0, sem_ld1, sem_ld2, sem_g0, sem_g1, sem_s0, sem_s1) = refs
  else:
    (p_hbm, es_hbm, ed_hbm, er_hbm, w1b_hbm, src_hbm, dst_hbm,
     acc0_hbm, acc1_hbm,
     idx2, dst2, rows2, es2, ed2, er2, w1b_v,
     acc_sh,
     sem_ld0, sem_ld1, sem_ld2, sem_g0, sem_g1, sem_s0, sem_s1) = refs
  sem_ld = (sem_ld0, sem_ld1, sem_ld2)
  sem_g = (sem_g0, sem_g1)
  sem_s = (sem_s0, sem_s1)

  cid = lax.axis_index("c")
  sid = lax.axis_index("s")
  wid = sid * NC + cid
  zv = jnp.zeros((16,), jnp.float32)

  def zero_rows(e, c):
    rows2[0, e, pl.ds(0, 16)] = zv
    rows2[0, e, pl.ds(16, 16)] = zv
    return c
  lax.fori_loop(0, RW, zero_rows, 0)
  rows_v = rows2.at[0, pl.ds(0, RW)]
  zbase = sid * 640

  @pl.when(sid < NS - 1)
  def _():
    for off in (0, 128, 256, 384, 512):
      pltpu.sync_copy(rows_v, acc_sh.at[pl.ds(zbase + off, 128)])

  @pl.when(sid == NS - 1)
  def _():
    for off in (0, 128, 256):
      pltpu.sync_copy(rows_v, acc_sh.at[pl.ds(zbase + off, 128)])
    pltpu.sync_copy(rows2.at[0, pl.ds(0, 16)],
                    acc_sh.at[pl.ds(zbase + 384, 16)])

  if with_deg:
    def zero_vz(i, c):
      vz_v[pl.ds(i * 16, 16)] = zv
      return c
    lax.fori_loop(0, 40, zero_vz, 0)
    dzb = sid * 640

    @pl.when(sid < NS - 1)
    def _():
      pltpu.sync_copy(vz_v, deg_sh.at[pl.ds(dzb, 640)])

    @pl.when(sid == NS - 1)
    def _():
      pltpu.sync_copy(vz_v.at[pl.ds(0, 400)], deg_sh.at[pl.ds(dzb, 400)])

    ov = jnp.ones((16,), jnp.float32)
    def fill_ones(i, c):
      ones_v[pl.ds(i * 16, 16)] = ov
      return c
    lax.fori_loop(0, 8, fill_ones, 0)

  pltpu.sync_copy(w1b_hbm, w1b_v)
  cvecs = [[w1b_v[k, pl.ds(h * 16, 16)] for h in range(2)] for k in range(3)]

  plsc.subcore_barrier()

  base_row = wid * RPT

  def issue_loads(b):
    r0 = base_row + b * G
    lb = b % 3
    return [
        pltpu.async_copy(src_hbm.at[pl.ds(r0, G)], idx2.at[lb], sem_ld[lb]),
        pltpu.async_copy(dst_hbm.at[pl.ds(r0, G)], dst2.at[lb], sem_ld[lb]),
        pltpu.async_copy(es_hbm.at[pl.ds(r0 * RW, GE)], es2.at[lb],
                         sem_ld[lb]),
        pltpu.async_copy(ed_hbm.at[pl.ds(r0 * RW, GE)], ed2.at[lb],
                         sem_ld[lb]),
        pltpu.async_copy(er_hbm.at[pl.ds(r0 * RW, GE)], er2.at[lb],
                         sem_ld[lb]),
    ]

  def issue_gathers(b):
    lb = b % 3
    rb = b % 2
    return [
        pltpu.async_copy(p_hbm.at[idx2.at[lb, j]],
                         rows2.at[rb, pl.ds(j * RW, RW)], sem_g[rb])
        for j in range(G)
    ]

  def compute_block(b, n_edges):
    lb = b % 3
    rb = b % 2

    @plsc.parallel_loop(0, n_edges, unroll=8)
    def _(e):
      eidx = jnp.full((16,), e, dtype=jnp.int32)
      se = plsc.load_gather(es2.at[lb], [eidx])
      sd = plsc.load_gather(ed2.at[lb], [eidx])
      sr = plsc.load_gather(er2.at[lb], [eidx])
      for h, off in ((0, 0), (1, 16)):
        wqv = se * cvecs[0][h] + sd * cvecs[1][h] + sr * cvecs[2][h]
        a = rows2[rb, e, pl.ds(off, 16)] + wqv
        rows2[rb, e, pl.ds(off, 16)] = jnp.maximum(a, a * 0.01)

  def issue_scatters(b):
    lb = b % 3
    rb = b % 2
    ds_ = [
        pltpu.async_copy(rows2.at[rb, pl.ds(j * RW, RW)],
                         acc_sh.at[dst2.at[lb, j]], sem_s[rb], add=True)
        for j in range(G)
    ]
    if with_deg:
      ds_ += [
          pltpu.async_copy(ones_v, deg_sh.at[dst2.at[lb, j]], sem_s[rb],
                           add=True)
          for j in range(G)
      ]
    return ds_

  ld = [None, None, None]
  g = [None, None]
  s = [None, None]
  ld[0] = issue_loads(0)
  for d in ld[0]:
    d.wait()
  g[0] = issue_gathers(0)
  ld[1] = issue_loads(1)
  for b in range(NBLK):
    rb = b % 2
    orb = 1 - rb
    for d in g[rb]:
      d.wait()
    if b + 1 < NBLK:
      for d in ld[(b + 1) % 3]:
        d.wait()
      if b >= 1:
        for d in s[orb]:
          d.wait()
      g[orb] = issue_gathers(b + 1)
    compute_block(b, GE)
    s[rb] = issue_scatters(b)
    if b + 2 < NBLK:
      ld[(b + 2) % 3] = issue_loads(b + 2)
  for d in s[0] + s[1]:
    d.wait()

  @pl.when(wid < ROWS - 32 * RPT)
  def _():
    trow = 32 * RPT + wid
    pltpu.async_copy(src_hbm.at[pl.ds(trow, 1)], idx2.at[0, pl.ds(0, 1)],
                     sem_ld[0]).wait()
    pltpu.async_copy(dst_hbm.at[pl.ds(trow, 1)], dst2.at[0, pl.ds(0, 1)],
                     sem_ld[0]).wait()
    pltpu.async_copy(es_hbm.at[pl.ds(trow * RW, RW)],
                     es2.at[0, pl.ds(0, RW)], sem_ld[0]).wait()
    pltpu.async_copy(ed_hbm.at[pl.ds(trow * RW, RW)],
                     ed2.at[0, pl.ds(0, RW)], sem_ld[0]).wait()
    pltpu.async_copy(er_hbm.at[pl.ds(trow * RW, RW)],
                     er2.at[0, pl.ds(0, RW)], sem_ld[0]).wait()
    pltpu.async_copy(p_hbm.at[idx2.at[0, 0]], rows2.at[0, pl.ds(0, RW)],
                     sem_g[0]).wait()
    compute_block(0, RW)
    pltpu.async_copy(rows2.at[0, pl.ds(0, RW)], acc_sh.at[dst2.at[0, 0]],
                     sem_s[0], add=True).wait()
    if with_deg:
      pltpu.async_copy(ones_v, deg_sh.at[dst2.at[0, 0]], sem_s[0],
                       add=True).wait()

  plsc.subcore_barrier()

  for cc, acc_hbm in ((0, acc0_hbm), (1, acc1_hbm)):
    @pl.when((sid < NS - 1) & (cid == cc))
    def _(acc_hbm=acc_hbm):
      pltpu.sync_copy(acc_sh.at[pl.ds(zbase, 640)],
                      acc_hbm.at[pl.ds(zbase, 640)])

    @pl.when((sid == NS - 1) & (cid == cc))
    def _(acc_hbm=acc_hbm):
      pltpu.sync_copy(acc_sh.at[pl.ds(zbase, 400)],
                      acc_hbm.at[pl.ds(zbase, 400)])

  if with_deg:
    dzb = sid * 640
    ln_tail = 400

    @pl.when((sid < NS - 1) & (cid == 0))
    def _():
      pltpu.sync_copy(deg_sh.at[pl.ds(dzb, 640)], deg0_hbm.at[pl.ds(dzb, 640)])

    @pl.when((sid == NS - 1) & (cid == 0))
    def _():
      pltpu.sync_copy(deg_sh.at[pl.ds(dzb, ln_tail)],
                      deg0_hbm.at[pl.ds(dzb, ln_tail)])

    @pl.when((sid < NS - 1) & (cid == 1))
    def _():
      pltpu.sync_copy(deg_sh.at[pl.ds(dzb, 640)], deg1_hbm.at[pl.ds(dzb, 640)])

    @pl.when((sid == NS - 1) & (cid == 1))
    def _():
      pltpu.sync_copy(deg_sh.at[pl.ds(dzb, ln_tail)],
                      deg1_hbm.at[pl.ds(dzb, ln_tail)])


_SC_MESH = plsc.VectorSubcoreMesh(
    core_axis_name="c", subcore_axis_name="s", num_cores=NC, num_subcores=NS)

_SC_PARAMS = pltpu.CompilerParams(use_tc_tiling_on_sc=False,
                                  needs_layout_passes=False)

_SC_SCRATCH_COMMON = [
    pltpu.VMEM((3, G, RW), jnp.int32),
    pltpu.VMEM((3, G, RW), jnp.int32),
    pltpu.VMEM((2, GE, D), jnp.float32),
    pltpu.VMEM((3, GE), jnp.float32),
    pltpu.VMEM((3, GE), jnp.float32),
    pltpu.VMEM((3, GE), jnp.float32),
    pltpu.VMEM((3, D), jnp.float32),
]

_SC_SEMS = [pltpu.SemaphoreType.DMA] * 7

_sc_edge_deg = pl.kernel(
    functools.partial(_sc_edge_body, True),
    out_type=(jax.ShapeDtypeStruct((N, D), jnp.float32),
              jax.ShapeDtypeStruct((N, D), jnp.float32),
              jax.ShapeDtypeStruct((N,), jnp.float32),
              jax.ShapeDtypeStruct((N,), jnp.float32)),
    mesh=_SC_MESH,
    scratch_types=_SC_SCRATCH_COMMON + [
        pltpu.VMEM((RW,), jnp.float32),
        pltpu.VMEM((640,), jnp.float32),
        pltpu.VMEM_SHARED((N, D), jnp.float32),
        pltpu.VMEM_SHARED((N,), jnp.float32),
    ] + _SC_SEMS,
    compiler_params=_SC_PARAMS,
    name="sc_edge_deg",
)

_sc_edge = pl.kernel(
    functools.partial(_sc_edge_body, False),
    out_type=(jax.ShapeDtypeStruct((N, D), jnp.float32),
              jax.ShapeDtypeStruct((N, D), jnp.float32)),
    mesh=_SC_MESH,
    scratch_types=_SC_SCRATCH_COMMON + [
        pltpu.VMEM_SHARED((N, D), jnp.float32),
    ] + _SC_SEMS,
    compiler_params=_SC_PARAMS,
    name="sc_edge",
)


def _prep_body(gate_ref, emb_ref, w1aT_ref, h0_ref, p0_ref):
  g = gate_ref[...]
  iota = lax.broadcasted_iota(jnp.int32, (NB, F), 1)
  oh = jnp.where(g == iota, 1.0, 0.0).astype(jnp.float32)
  h0 = _dot(oh, emb_ref[...])
  h0_ref[...] = h0
  p0_ref[...] = _dot(h0, w1aT_ref[...])


_prep = pl.pallas_call(
    _prep_body,
    grid=(GRID_N,),
    in_specs=[
        pl.BlockSpec((NB, 1), lambda i: (i, 0)),
        pl.BlockSpec((F, F), lambda i: (0, 0)),
        pl.BlockSpec((F, D), lambda i: (0, 0)),
    ],
    out_specs=(pl.BlockSpec((NB, F), lambda i: (i, 0)),
               pl.BlockSpec((NB, D), lambda i: (i, 0))),
    out_shape=(jax.ShapeDtypeStruct((N, F), jnp.float32),
               jax.ShapeDtypeStruct((N, D), jnp.float32)),
)


def _rdeg_body(d0_ref, d1_ref, rd_ref):
  rd_ref[...] = 1.0 / jnp.maximum(d0_ref[...] + d1_ref[...], 1.0)


_rdeg = pl.pallas_call(
    _rdeg_body,
    grid=(GRID_N,),
    in_specs=[
        pl.BlockSpec((NB, 1), lambda i: (i, 0)),
        pl.BlockSpec((NB, 1), lambda i: (i, 0)),
    ],
    out_specs=pl.BlockSpec((NB, 1), lambda i: (i, 0)),
    out_shape=jax.ShapeDtypeStruct((N, 1), jnp.float32),
)


def _layer_body(h_ref, a0_ref, a1_ref, rd_ref, w2aT_ref, w2bT_ref,
                b2_ref, w1aT_ref, hout_ref, pout_ref):
  hN = (a0_ref[...] + a1_ref[...]) * rd_ref[...]
  z = _dot(h_ref[...], w2aT_ref[...]) + _dot(hN, w2bT_ref[...]) + b2_ref[...]
  hn = jnp.maximum(z, 0.0)
  hout_ref[...] = hn
  pout_ref[...] = _dot(hn, w1aT_ref[...])


_layer = pl.pallas_call(
    _layer_body,
    grid=(GRID_N,),
    in_specs=[
        pl.BlockSpec((NB, F), lambda i: (i, 0)),
        pl.BlockSpec((NB, D), lambda i: (i, 0)),
        pl.BlockSpec((NB, D), lambda i: (i, 0)),
        pl.BlockSpec((NB, 1), lambda i: (i, 0)),
        pl.BlockSpec((F, F), lambda i: (0, 0)),
        pl.BlockSpec((D, F), lambda i: (0, 0)),
        pl.BlockSpec((1, F), lambda i: (0, 0)),
        pl.BlockSpec((F, D), lambda i: (0, 0)),
    ],
    out_specs=(pl.BlockSpec((NB, F), lambda i: (i, 0)),
               pl.BlockSpec((NB, D), lambda i: (i, 0))),
    out_shape=(jax.ShapeDtypeStruct((N, F), jnp.float32),
               jax.ShapeDtypeStruct((N, D), jnp.float32)),
)


def _final_body(h_ref, a0_ref, a1_ref, rd_ref, w2aT_ref, w2bT_ref,
                b2_ref, l1wT_ref, l1b_ref, l2wT_ref, l2b_ref, out_ref):
  hN = (a0_ref[...] + a1_ref[...]) * rd_ref[...]
  z = _dot(h_ref[...], w2aT_ref[...]) + _dot(hN, w2bT_ref[...]) + b2_ref[...]
  hn = jnp.maximum(z, 0.0)
  h1 = jnp.maximum(_dot(hn, l1wT_ref[...]) + l1b_ref[...], 0.0)
  out_ref[...] = _dot(h1, l2wT_ref[...]) + l2b_ref[...]


_final = pl.pallas_call(
    _final_body,
    grid=(GRID_N,),
    in_specs=[
        pl.BlockSpec((NB, F), lambda i: (i, 0)),
        pl.BlockSpec((NB, D), lambda i: (i, 0)),
        pl.BlockSpec((NB, D), lambda i: (i, 0)),
        pl.BlockSpec((NB, 1), lambda i: (i, 0)),
        pl.BlockSpec((F, F), lambda i: (0, 0)),
        pl.BlockSpec((D, F), lambda i: (0, 0)),
        pl.BlockSpec((1, F), lambda i: (0, 0)),
        pl.BlockSpec((F, F), lambda i: (0, 0)),
        pl.BlockSpec((1, F), lambda i: (0, 0)),
        pl.BlockSpec((F, D), lambda i: (0, 0)),
        pl.BlockSpec((1, D), lambda i: (0, 0)),
    ],
    out_specs=pl.BlockSpec((NB, D), lambda i: (i, 0)),
    out_shape=jax.ShapeDtypeStruct((N, D), jnp.float32),
)


def kernel(gate_type, edge_index, edge_src_idx, edge_dst_idx, edge_reversed,
           emb, W1_0, W2_0, b2_0, W1_1, W2_1, b2_1, W1_2, W2_2, b2_2,
           W1_3, W2_3, b2_3, W1_4, W2_4, b2_4, L1W, L1b, L2W, L2b):
  gate2d = gate_type.astype(jnp.int32).reshape(N, 1)
  src2d = edge_index[0].astype(jnp.int32).reshape(ROWS, RW)
  dst2d = edge_index[1].astype(jnp.int32).reshape(ROWS, RW)

  convs = [(W1_0, W2_0, b2_0), (W1_1, W2_1, b2_1), (W1_2, W2_2, b2_2),
           (W1_3, W2_3, b2_3), (W1_4, W2_4, b2_4)]
  W1aT, W1bT, W2aT, W2bT, b2r = [], [], [], [], []
  for W1, W2, b2 in convs:
    W1aT.append(W1[:, :F].T)
    W1bT.append(W1[:, F:F + 3].T)
    W2aT.append(W2[:, :F].T)
    W2bT.append(W2[:, F:].T)
    b2r.append(b2.reshape(1, F))

  h, p = _prep(gate2d, emb, W1aT[0])

  out = None
  rdeg = None
  for l in range(5):
    if l == 0:
      a0, a1, deg0, deg1 = _sc_edge_deg(
          p, edge_src_idx, edge_dst_idx, edge_reversed, W1bT[l], src2d, dst2d)
      rdeg = _rdeg(deg0.reshape(N, 1), deg1.reshape(N, 1))
    else:
      a0, a1 = _sc_edge(
          p, edge_src_idx, edge_dst_idx, edge_reversed, W1bT[l], src2d, dst2d)
    if l < 4:
      h, p = _layer(h, a0, a1, rdeg, W2aT[l], W2bT[l], b2r[l], W1aT[l + 1])
    else:
      out = _final(h, a0, a1, rdeg, W2aT[4], W2bT[4], b2r[4],
                   L1W.T, L1b.reshape(1, F), L2W.T, L2b.reshape(1, D))
  return out


# speedup vs baseline: 13.4667x; 1.0163x over previous
import functools

import jax
import jax.numpy as jnp
from jax import lax
from jax.experimental import pallas as pl
from jax.experimental.pallas import tpu as pltpu
from jax.experimental.pallas import tpu_sc as plsc

N = 10000
E = 320000
F = 128
D = 32
NC, NS = 2, 16
RW = 128
ROWS = E // RW
ROWS_PER_SC = ROWS // NC
NPS = N // NS
NB = 2000
GRID_N = N // NB
EB = 6400
GRID_E = E // EB

_HI = lax.Precision.HIGHEST


def _dot(a, b):
  return jnp.dot(a, b, preferred_element_type=jnp.float32, precision=_HI)


G = 6
NBLK = 13
RPT = G * NBLK
GE = G * RW


def _sc_edge_body(with_deg, *refs):
  if with_deg:
    (p_hbm, es_hbm, ed_hbm, er_hbm, w1b_hbm, src_hbm, dst_hbm,
     acc0_hbm, acc1_hbm, deg0_hbm, deg1_hbm,
     idx2, dst2, rows2, es2, ed2, er2, w1b_v, ones_v, vz_v,
     acc_sh, deg_sh,
     sem_ld0, sem_ld1, sem_ld2, sem_ld3,
     sem_g0, sem_g1, sem_g2, sem_s0, sem_s1, sem_s2) = refs
  else:
    (p_hbm, es_hbm, ed_hbm, er_hbm, w1b_hbm, src_hbm, dst_hbm,
     acc0_hbm, acc1_hbm,
     idx2, dst2, rows2, es2, ed2, er2, w1b_v,
     acc_sh,
     sem_ld0, sem_ld1, sem_ld2, sem_ld3,
     sem_g0, sem_g1, sem_g2, sem_s0, sem_s1, sem_s2) = refs
  sem_ld = (sem_ld0, sem_ld1, sem_ld2, sem_ld3)
  sem_g = (sem_g0, sem_g1, sem_g2)
  sem_s = (sem_s0, sem_s1, sem_s2)

  cid = lax.axis_index("c")
  sid = lax.axis_index("s")
  wid = sid * NC + cid
  zv = jnp.zeros((16,), jnp.float32)

  def zero_rows(e, c):
    rows2[0, e, pl.ds(0, 16)] = zv
    rows2[0, e, pl.ds(16, 16)] = zv
    return c
  lax.fori_loop(0, RW, zero_rows, 0)
  rows_v = rows2.at[0, pl.ds(0, RW)]
  zbase = sid * 640

  @pl.when(sid < NS - 1)
  def _():
    for off in (0, 128, 256, 384, 512):
      pltpu.sync_copy(rows_v, acc_sh.at[pl.ds(zbase + off, 128)])

  @pl.when(sid == NS - 1)
  def _():
    for off in (0, 128, 256):
      pltpu.sync_copy(rows_v, acc_sh.at[pl.ds(zbase + off, 128)])
    pltpu.sync_copy(rows2.at[0, pl.ds(0, 16)],
                    acc_sh.at[pl.ds(zbase + 384, 16)])

  if with_deg:
    def zero_vz(i, c):
      vz_v[pl.ds(i * 16, 16)] = zv
      return c
    lax.fori_loop(0, 40, zero_vz, 0)
    dzb = sid * 640

    @pl.when(sid < NS - 1)
    def _():
      pltpu.sync_copy(vz_v, deg_sh.at[pl.ds(dzb, 640)])

    @pl.when(sid == NS - 1)
    def _():
      pltpu.sync_copy(vz_v.at[pl.ds(0, 400)], deg_sh.at[pl.ds(dzb, 400)])

    ov = jnp.ones((16,), jnp.float32)
    def fill_ones(i, c):
      ones_v[pl.ds(i * 16, 16)] = ov
      return c
    lax.fori_loop(0, 8, fill_ones, 0)

  pltpu.sync_copy(w1b_hbm, w1b_v)
  cvecs = [[w1b_v[k, pl.ds(h * 16, 16)] for h in range(2)] for k in range(3)]

  plsc.subcore_barrier()

  base_row = wid * RPT

  def issue_loads(b):
    r0 = base_row + b * G
    lb = b % 4
    return [
        pltpu.async_copy(src_hbm.at[pl.ds(r0, G)], idx2.at[lb], sem_ld[lb]),
        pltpu.async_copy(dst_hbm.at[pl.ds(r0, G)], dst2.at[lb], sem_ld[lb]),
        pltpu.async_copy(es_hbm.at[pl.ds(r0 * RW, GE)], es2.at[lb],
                         sem_ld[lb]),
        pltpu.async_copy(ed_hbm.at[pl.ds(r0 * RW, GE)], ed2.at[lb],
                         sem_ld[lb]),
        pltpu.async_copy(er_hbm.at[pl.ds(r0 * RW, GE)], er2.at[lb],
                         sem_ld[lb]),
    ]

  def issue_gathers(b):
    lb = b % 4
    rb = b % 3
    return [
        pltpu.async_copy(p_hbm.at[idx2.at[lb, j]],
                         rows2.at[rb, pl.ds(j * RW, RW)], sem_g[rb])
        for j in range(G)
    ]

  def compute_block(b, n_edges):
    lb = b % 4
    rb = b % 3

    @plsc.parallel_loop(0, n_edges, unroll=8)
    def _(e):
      eidx = jnp.full((16,), e, dtype=jnp.int32)
      se = plsc.load_gather(es2.at[lb], [eidx])
      sd = plsc.load_gather(ed2.at[lb], [eidx])
      sr = plsc.load_gather(er2.at[lb], [eidx])
      for h, off in ((0, 0), (1, 16)):
        wqv = se * cvecs[0][h] + sd * cvecs[1][h] + sr * cvecs[2][h]
        a = rows2[rb, e, pl.ds(off, 16)] + wqv
        rows2[rb, e, pl.ds(off, 16)] = jnp.maximum(a, a * 0.01)

  def issue_scatters(b):
    lb = b % 4
    rb = b % 3
    ds_ = [
        pltpu.async_copy(rows2.at[rb, pl.ds(j * RW, RW)],
                         acc_sh.at[dst2.at[lb, j]], sem_s[rb], add=True)
        for j in range(G)
    ]
    if with_deg:
      ds_ += [
          pltpu.async_copy(ones_v, deg_sh.at[dst2.at[lb, j]], sem_s[rb],
                           add=True)
          for j in range(G)
      ]
    return ds_

  ld = [None] * 4
  g = [None] * 3
  s = [None] * 3
  ld[0] = issue_loads(0)
  for d in ld[0]:
    d.wait()
  g[0] = issue_gathers(0)
  ld[1] = issue_loads(1)
  for b in range(NBLK):
    if b + 1 < NBLK:
      for d in ld[(b + 1) % 4]:
        d.wait()
      if b >= 2:
        for d in s[(b + 1) % 3]:
          d.wait()
      g[(b + 1) % 3] = issue_gathers(b + 1)
    for d in g[b % 3]:
      d.wait()
    compute_block(b, GE)
    s[b % 3] = issue_scatters(b)
    if b + 2 < NBLK:
      ld[(b + 2) % 4] = issue_loads(b + 2)
  for bb in (NBLK - 2, NBLK - 1):
    for d in s[bb % 3]:
      d.wait()

  @pl.when(wid < ROWS - 32 * RPT)
  def _():
    trow = 32 * RPT + wid
    pltpu.async_copy(src_hbm.at[pl.ds(trow, 1)], idx2.at[0, pl.ds(0, 1)],
                     sem_ld[0]).wait()
    pltpu.async_copy(dst_hbm.at[pl.ds(trow, 1)], dst2.at[0, pl.ds(0, 1)],
                     sem_ld[0]).wait()
    pltpu.async_copy(es_hbm.at[pl.ds(trow * RW, RW)],
                     es2.at[0, pl.ds(0, RW)], sem_ld[0]).wait()
    pltpu.async_copy(ed_hbm.at[pl.ds(trow * RW, RW)],
                     ed2.at[0, pl.ds(0, RW)], sem_ld[0]).wait()
    pltpu.async_copy(er_hbm.at[pl.ds(trow * RW, RW)],
                     er2.at[0, pl.ds(0, RW)], sem_ld[0]).wait()
    pltpu.async_copy(p_hbm.at[idx2.at[0, 0]], rows2.at[0, pl.ds(0, RW)],
                     sem_g[0]).wait()
    compute_block(0, RW)
    pltpu.async_copy(rows2.at[0, pl.ds(0, RW)], acc_sh.at[dst2.at[0, 0]],
                     sem_s[0], add=True).wait()
    if with_deg:
      pltpu.async_copy(ones_v, deg_sh.at[dst2.at[0, 0]], sem_s[0],
                       add=True).wait()

  plsc.subcore_barrier()

  for cc, acc_hbm in ((0, acc0_hbm), (1, acc1_hbm)):
    @pl.when((sid < NS - 1) & (cid == cc))
    def _(acc_hbm=acc_hbm):
      pltpu.sync_copy(acc_sh.at[pl.ds(zbase, 640)],
                      acc_hbm.at[pl.ds(zbase, 640)])

    @pl.when((sid == NS - 1) & (cid == cc))
    def _(acc_hbm=acc_hbm):
      pltpu.sync_copy(acc_sh.at[pl.ds(zbase, 400)],
                      acc_hbm.at[pl.ds(zbase, 400)])

  if with_deg:
    dzb = sid * 640
    ln_tail = 400

    @pl.when((sid < NS - 1) & (cid == 0))
    def _():
      pltpu.sync_copy(deg_sh.at[pl.ds(dzb, 640)], deg0_hbm.at[pl.ds(dzb, 640)])

    @pl.when((sid == NS - 1) & (cid == 0))
    def _():
      pltpu.sync_copy(deg_sh.at[pl.ds(dzb, ln_tail)],
                      deg0_hbm.at[pl.ds(dzb, ln_tail)])

    @pl.when((sid < NS - 1) & (cid == 1))
    def _():
      pltpu.sync_copy(deg_sh.at[pl.ds(dzb, 640)], deg1_hbm.at[pl.ds(dzb, 640)])

    @pl.when((sid == NS - 1) & (cid == 1))
    def _():
      pltpu.sync_copy(deg_sh.at[pl.ds(dzb, ln_tail)],
                      deg1_hbm.at[pl.ds(dzb, ln_tail)])


_SC_MESH = plsc.VectorSubcoreMesh(
    core_axis_name="c", subcore_axis_name="s", num_cores=NC, num_subcores=NS)

_SC_PARAMS = pltpu.CompilerParams(use_tc_tiling_on_sc=False,
                                  needs_layout_passes=False)

_SC_SCRATCH_COMMON = [
    pltpu.VMEM((4, G, RW), jnp.int32),
    pltpu.VMEM((4, G, RW), jnp.int32),
    pltpu.VMEM((3, GE, D), jnp.float32),
    pltpu.VMEM((4, GE), jnp.float32),
    pltpu.VMEM((4, GE), jnp.float32),
    pltpu.VMEM((4, GE), jnp.float32),
    pltpu.VMEM((3, D), jnp.float32),
]

_SC_SEMS = [pltpu.SemaphoreType.DMA] * 10

_sc_edge_deg = pl.kernel(
    functools.partial(_sc_edge_body, True),
    out_type=(jax.ShapeDtypeStruct((N, D), jnp.float32),
              jax.ShapeDtypeStruct((N, D), jnp.float32),
              jax.ShapeDtypeStruct((N,), jnp.float32),
              jax.ShapeDtypeStruct((N,), jnp.float32)),
    mesh=_SC_MESH,
    scratch_types=_SC_SCRATCH_COMMON + [
        pltpu.VMEM((RW,), jnp.float32),
        pltpu.VMEM((640,), jnp.float32),
        pltpu.VMEM_SHARED((N, D), jnp.float32),
        pltpu.VMEM_SHARED((N,), jnp.float32),
    ] + _SC_SEMS,
    compiler_params=_SC_PARAMS,
    name="sc_edge_deg",
)

_sc_edge = pl.kernel(
    functools.partial(_sc_edge_body, False),
    out_type=(jax.ShapeDtypeStruct((N, D), jnp.float32),
              jax.ShapeDtypeStruct((N, D), jnp.float32)),
    mesh=_SC_MESH,
    scratch_types=_SC_SCRATCH_COMMON + [
        pltpu.VMEM_SHARED((N, D), jnp.float32),
    ] + _SC_SEMS,
    compiler_params=_SC_PARAMS,
    name="sc_edge",
)


def _prep_body(gate_ref, emb_ref, w1aT_ref, h0_ref, p0_ref):
  g = gate_ref[...]
  iota = lax.broadcasted_iota(jnp.int32, (NB, F), 1)
  oh = jnp.where(g == iota, 1.0, 0.0).astype(jnp.float32)
  h0 = _dot(oh, emb_ref[...])
  h0_ref[...] = h0
  p0_ref[...] = _dot(h0, w1aT_ref[...])


_prep = pl.pallas_call(
    _prep_body,
    grid=(GRID_N,),
    in_specs=[
        pl.BlockSpec((NB, 1), lambda i: (i, 0)),
        pl.BlockSpec((F, F), lambda i: (0, 0)),
        pl.BlockSpec((F, D), lambda i: (0, 0)),
    ],
    out_specs=(pl.BlockSpec((NB, F), lambda i: (i, 0)),
               pl.BlockSpec((NB, D), lambda i: (i, 0))),
    out_shape=(jax.ShapeDtypeStruct((N, F), jnp.float32),
               jax.ShapeDtypeStruct((N, D), jnp.float32)),
)


def _rdeg_body(d0_ref, d1_ref, rd_ref):
  rd_ref[...] = 1.0 / jnp.maximum(d0_ref[...] + d1_ref[...], 1.0)


_rdeg = pl.pallas_call(
    _rdeg_body,
    grid=(GRID_N,),
    in_specs=[
        pl.BlockSpec((NB, 1), lambda i: (i, 0)),
        pl.BlockSpec((NB, 1), lambda i: (i, 0)),
    ],
    out_specs=pl.BlockSpec((NB, 1), lambda i: (i, 0)),
    out_shape=jax.ShapeDtypeStruct((N, 1), jnp.float32),
)


def _layer_body(h_ref, a0_ref, a1_ref, rd_ref, w2aT_ref, w2bT_ref,
                b2_ref, w1aT_ref, hout_ref, pout_ref):
  hN = (a0_ref[...] + a1_ref[...]) * rd_ref[...]
  z = _dot(h_ref[...], w2aT_ref[...]) + _dot(hN, w2bT_ref[...]) + b2_ref[...]
  hn = jnp.maximum(z, 0.0)
  hout_ref[...] = hn
  pout_ref[...] = _dot(hn, w1aT_ref[...])


_layer = pl.pallas_call(
    _layer_body,
    grid=(GRID_N,),
    in_specs=[
        pl.BlockSpec((NB, F), lambda i: (i, 0)),
        pl.BlockSpec((NB, D), lambda i: (i, 0)),
        pl.BlockSpec((NB, D), lambda i: (i, 0)),
        pl.BlockSpec((NB, 1), lambda i: (i, 0)),
        pl.BlockSpec((F, F), lambda i: (0, 0)),
        pl.BlockSpec((D, F), lambda i: (0, 0)),
        pl.BlockSpec((1, F), lambda i: (0, 0)),
        pl.BlockSpec((F, D), lambda i: (0, 0)),
    ],
    out_specs=(pl.BlockSpec((NB, F), lambda i: (i, 0)),
               pl.BlockSpec((NB, D), lambda i: (i, 0))),
    out_shape=(jax.ShapeDtypeStruct((N, F), jnp.float32),
               jax.ShapeDtypeStruct((N, D), jnp.float32)),
)


def _final_body(h_ref, a0_ref, a1_ref, rd_ref, w2aT_ref, w2bT_ref,
                b2_ref, l1wT_ref, l1b_ref, l2wT_ref, l2b_ref, out_ref):
  hN = (a0_ref[...] + a1_ref[...]) * rd_ref[...]
  z = _dot(h_ref[...], w2aT_ref[...]) + _dot(hN, w2bT_ref[...]) + b2_ref[...]
  hn = jnp.maximum(z, 0.0)
  h1 = jnp.maximum(_dot(hn, l1wT_ref[...]) + l1b_ref[...], 0.0)
  out_ref[...] = _dot(h1, l2wT_ref[...]) + l2b_ref[...]


_final = pl.pallas_call(
    _final_body,
    grid=(GRID_N,),
    in_specs=[
        pl.BlockSpec((NB, F), lambda i: (i, 0)),
        pl.BlockSpec((NB, D), lambda i: (i, 0)),
        pl.BlockSpec((NB, D), lambda i: (i, 0)),
        pl.BlockSpec((NB, 1), lambda i: (i, 0)),
        pl.BlockSpec((F, F), lambda i: (0, 0)),
        pl.BlockSpec((D, F), lambda i: (0, 0)),
        pl.BlockSpec((1, F), lambda i: (0, 0)),
        pl.BlockSpec((F, F), lambda i: (0, 0)),
        pl.BlockSpec((1, F), lambda i: (0, 0)),
        pl.BlockSpec((F, D), lambda i: (0, 0)),
        pl.BlockSpec((1, D), lambda i: (0, 0)),
    ],
    out_specs=pl.BlockSpec((NB, D), lambda i: (i, 0)),
    out_shape=jax.ShapeDtypeStruct((N, D), jnp.float32),
)


def kernel(gate_type, edge_index, edge_src_idx, edge_dst_idx, edge_reversed,
           emb, W1_0, W2_0, b2_0, W1_1, W2_1, b2_1, W1_2, W2_2, b2_2,
           W1_3, W2_3, b2_3, W1_4, W2_4, b2_4, L1W, L1b, L2W, L2b):
  gate2d = gate_type.astype(jnp.int32).reshape(N, 1)
  src2d = edge_index[0].astype(jnp.int32).reshape(ROWS, RW)
  dst2d = edge_index[1].astype(jnp.int32).reshape(ROWS, RW)

  convs = [(W1_0, W2_0, b2_0), (W1_1, W2_1, b2_1), (W1_2, W2_2, b2_2),
           (W1_3, W2_3, b2_3), (W1_4, W2_4, b2_4)]
  W1aT, W1bT, W2aT, W2bT, b2r = [], [], [], [], []
  for W1, W2, b2 in convs:
    W1aT.append(W1[:, :F].T)
    W1bT.append(W1[:, F:F + 3].T)
    W2aT.append(W2[:, :F].T)
    W2bT.append(W2[:, F:].T)
    b2r.append(b2.reshape(1, F))

  h, p = _prep(gate2d, emb, W1aT[0])

  out = None
  rdeg = None
  for l in range(5):
    if l == 0:
      a0, a1, deg0, deg1 = _sc_edge_deg(
          p, edge_src_idx, edge_dst_idx, edge_reversed, W1bT[l], src2d, dst2d)
      rdeg = _rdeg(deg0.reshape(N, 1), deg1.reshape(N, 1))
    else:
      a0, a1 = _sc_edge(
          p, edge_src_idx, edge_dst_idx, edge_reversed, W1bT[l], src2d, dst2d)
    if l < 4:
      h, p = _layer(h, a0, a1, rdeg, W2aT[l], W2bT[l], b2r[l], W1aT[l + 1])
    else:
      out = _final(h, a0, a1, rdeg, W2aT[4], W2bT[4], b2r[4],
                   L1W.T, L1b.reshape(1, F), L2W.T, L2b.reshape(1, D))
  return out


# speedup vs baseline: 14.0192x; 1.0410x over previous
import functools

import jax
import jax.numpy as jnp
from jax import lax
from jax.experimental import pallas as pl
from jax.experimental.pallas import tpu as pltpu
from jax.experimental.pallas import tpu_sc as plsc

N = 10000
E = 320000
F = 128
D = 32
NC, NS = 2, 16
RW = 128
ROWS = E // RW
ROWS_PER_SC = ROWS // NC
NPS = N // NS
NB = 2000
GRID_N = N // NB
EB = 6400
GRID_E = E // EB

_HI = lax.Precision.HIGHEST


def _dot(a, b):
  return jnp.dot(a, b, preferred_element_type=jnp.float32, precision=_HI)


G = 6
NBLK = 13
RPT = G * NBLK
GE = G * RW


def _sc_edge_body(with_deg, *refs):
  if with_deg:
    (p_hbm, es_hbm, ed_hbm, er_hbm, w1b_hbm, src_hbm, dst_hbm,
     acc0_hbm, acc1_hbm, deg0_hbm, deg1_hbm,
     idx2, dst2, rows2, es2, ed2, er2, w1b_v, ones_v, vz_v,
     acc_sh, deg_sh,
     sem_ld0, sem_ld1, sem_ld2, sem_ld3,
     sem_g0, sem_g1, sem_g2, sem_s0, sem_s1, sem_s2) = refs
  else:
    (p_hbm, es_hbm, ed_hbm, er_hbm, w1b_hbm, src_hbm, dst_hbm,
     acc0_hbm, acc1_hbm,
     idx2, dst2, rows2, es2, ed2, er2, w1b_v,
     acc_sh,
     sem_ld0, sem_ld1, sem_ld2, sem_ld3,
     sem_g0, sem_g1, sem_g2, sem_s0, sem_s1, sem_s2) = refs
  sem_ld = (sem_ld0, sem_ld1, sem_ld2, sem_ld3)
  sem_g = (sem_g0, sem_g1, sem_g2)
  sem_s = (sem_s0, sem_s1, sem_s2)

  cid = lax.axis_index("c")
  sid = lax.axis_index("s")
  wid = sid * NC + cid
  zv = jnp.zeros((16,), jnp.float32)

  def zero_rows(e, c):
    rows2[0, e, pl.ds(0, 16)] = zv
    rows2[0, e, pl.ds(16, 16)] = zv
    return c
  lax.fori_loop(0, RW, zero_rows, 0)
  rows_v = rows2.at[0, pl.ds(0, RW)]
  zbase = sid * 640

  @pl.when(sid < NS - 1)
  def _():
    for off in (0, 128, 256, 384, 512):
      pltpu.sync_copy(rows_v, acc_sh.at[pl.ds(zbase + off, 128)])

  @pl.when(sid == NS - 1)
  def _():
    for off in (0, 128, 256):
      pltpu.sync_copy(rows_v, acc_sh.at[pl.ds(zbase + off, 128)])
    pltpu.sync_copy(rows2.at[0, pl.ds(0, 16)],
                    acc_sh.at[pl.ds(zbase + 384, 16)])

  if with_deg:
    def zero_vz(i, c):
      vz_v[pl.ds(i * 16, 16)] = zv
      return c
    lax.fori_loop(0, 40, zero_vz, 0)
    dzb = sid * 640

    @pl.when(sid < NS - 1)
    def _():
      pltpu.sync_copy(vz_v, deg_sh.at[pl.ds(dzb, 640)])

    @pl.when(sid == NS - 1)
    def _():
      pltpu.sync_copy(vz_v.at[pl.ds(0, 400)], deg_sh.at[pl.ds(dzb, 400)])

    ov = jnp.ones((16,), jnp.float32)
    def fill_ones(i, c):
      ones_v[pl.ds(i * 16, 16)] = ov
      return c
    lax.fori_loop(0, 8, fill_ones, 0)

  pltpu.sync_copy(w1b_hbm, w1b_v)
  cvecs = [[w1b_v[k, pl.ds(h * 16, 16)] for h in range(2)] for k in range(3)]

  plsc.subcore_barrier()

  base_row = wid * RPT

  def issue_loads(b):
    r0 = base_row + b * G
    lb = b % 4
    return [
        pltpu.async_copy(src_hbm.at[pl.ds(r0, G)], idx2.at[lb], sem_ld[lb]),
        pltpu.async_copy(dst_hbm.at[pl.ds(r0, G)], dst2.at[lb], sem_ld[lb]),
        pltpu.async_copy(es_hbm.at[pl.ds(r0 * RW, GE)], es2.at[lb],
                         sem_ld[lb]),
        pltpu.async_copy(ed_hbm.at[pl.ds(r0 * RW, GE)], ed2.at[lb],
                         sem_ld[lb]),
        pltpu.async_copy(er_hbm.at[pl.ds(r0 * RW, GE)], er2.at[lb],
                         sem_ld[lb]),
    ]

  def issue_gathers(b):
    lb = b % 4
    rb = b % 3
    return [
        pltpu.async_copy(p_hbm.at[idx2.at[lb, j]],
                         rows2.at[rb, pl.ds(j * RW, RW)], sem_g[rb])
        for j in range(G)
    ]

  def compute_block(b, n_edges):
    lb = b % 4
    rb = b % 3

    @plsc.parallel_loop(0, n_edges, unroll=8)
    def _(e):
      eidx = jnp.full((16,), e, dtype=jnp.int32)
      se = plsc.load_gather(es2.at[lb], [eidx])
      sd = plsc.load_gather(ed2.at[lb], [eidx])
      sr = plsc.load_gather(er2.at[lb], [eidx])
      for h, off in ((0, 0), (1, 16)):
        wqv = se * cvecs[0][h] + sd * cvecs[1][h] + sr * cvecs[2][h]
        a = rows2[rb, e, pl.ds(off, 16)] + wqv
        rows2[rb, e, pl.ds(off, 16)] = jnp.maximum(a, a * 0.01)

  def issue_scatters(b):
    lb = b % 4
    rb = b % 3
    ds_ = [
        pltpu.async_copy(rows2.at[rb, pl.ds(j * RW, RW)],
                         acc_sh.at[dst2.at[lb, j]], sem_s[rb], add=True)
        for j in range(G)
    ]
    if with_deg:
      ds_ += [
          pltpu.async_copy(ones_v, deg_sh.at[dst2.at[lb, j]], sem_s[rb],
                           add=True)
          for j in range(G)
      ]
    return ds_

  ld = [None] * 4
  g = [None] * 3
  s = [None] * 3
  ld[0] = issue_loads(0)
  for d in ld[0]:
    d.wait()
  g[0] = issue_gathers(0)
  ld[1] = issue_loads(1)
  for b in range(NBLK):
    if b + 1 < NBLK:
      for d in ld[(b + 1) % 4]:
        d.wait()
      if b >= 2:
        for d in s[(b + 1) % 3]:
          d.wait()
      g[(b + 1) % 3] = issue_gathers(b + 1)
    for d in g[b % 3]:
      d.wait()
    compute_block(b, GE)
    s[b % 3] = issue_scatters(b)
    if b + 2 < NBLK:
      ld[(b + 2) % 4] = issue_loads(b + 2)
  for bb in (NBLK - 2, NBLK - 1):
    for d in s[bb % 3]:
      d.wait()

  @pl.when(wid < ROWS - 32 * RPT)
  def _():
    trow = 32 * RPT + wid
    pltpu.async_copy(src_hbm.at[pl.ds(trow, 1)], idx2.at[0, pl.ds(0, 1)],
                     sem_ld[0]).wait()
    pltpu.async_copy(dst_hbm.at[pl.ds(trow, 1)], dst2.at[0, pl.ds(0, 1)],
                     sem_ld[0]).wait()
    pltpu.async_copy(es_hbm.at[pl.ds(trow * RW, RW)],
                     es2.at[0, pl.ds(0, RW)], sem_ld[0]).wait()
    pltpu.async_copy(ed_hbm.at[pl.ds(trow * RW, RW)],
                     ed2.at[0, pl.ds(0, RW)], sem_ld[0]).wait()
    pltpu.async_copy(er_hbm.at[pl.ds(trow * RW, RW)],
                     er2.at[0, pl.ds(0, RW)], sem_ld[0]).wait()
    pltpu.async_copy(p_hbm.at[idx2.at[0, 0]], rows2.at[0, pl.ds(0, RW)],
                     sem_g[0]).wait()
    compute_block(0, RW)
    pltpu.async_copy(rows2.at[0, pl.ds(0, RW)], acc_sh.at[dst2.at[0, 0]],
                     sem_s[0], add=True).wait()
    if with_deg:
      pltpu.async_copy(ones_v, deg_sh.at[dst2.at[0, 0]], sem_s[0],
                       add=True).wait()

  plsc.subcore_barrier()

  for cc, acc_hbm in ((0, acc0_hbm), (1, acc1_hbm)):
    @pl.when((sid < NS - 1) & (cid == cc))
    def _(acc_hbm=acc_hbm):
      pltpu.sync_copy(acc_sh.at[pl.ds(zbase, 640)],
                      acc_hbm.at[pl.ds(zbase, 640)])

    @pl.when((sid == NS - 1) & (cid == cc))
    def _(acc_hbm=acc_hbm):
      pltpu.sync_copy(acc_sh.at[pl.ds(zbase, 400)],
                      acc_hbm.at[pl.ds(zbase, 400)])

  if with_deg:
    dzb = sid * 640
    ln_tail = 400

    @pl.when((sid < NS - 1) & (cid == 0))
    def _():
      pltpu.sync_copy(deg_sh.at[pl.ds(dzb, 640)], deg0_hbm.at[pl.ds(dzb, 640)])

    @pl.when((sid == NS - 1) & (cid == 0))
    def _():
      pltpu.sync_copy(deg_sh.at[pl.ds(dzb, ln_tail)],
                      deg0_hbm.at[pl.ds(dzb, ln_tail)])

    @pl.when((sid < NS - 1) & (cid == 1))
    def _():
      pltpu.sync_copy(deg_sh.at[pl.ds(dzb, 640)], deg1_hbm.at[pl.ds(dzb, 640)])

    @pl.when((sid == NS - 1) & (cid == 1))
    def _():
      pltpu.sync_copy(deg_sh.at[pl.ds(dzb, ln_tail)],
                      deg1_hbm.at[pl.ds(dzb, ln_tail)])


_SC_MESH = plsc.VectorSubcoreMesh(
    core_axis_name="c", subcore_axis_name="s", num_cores=NC, num_subcores=NS)

_SC_PARAMS = pltpu.CompilerParams(use_tc_tiling_on_sc=False,
                                  needs_layout_passes=False)

_SC_SCRATCH_COMMON = [
    pltpu.VMEM((4, G, RW), jnp.int32),
    pltpu.VMEM((4, G, RW), jnp.int32),
    pltpu.VMEM((3, GE, D), jnp.float32),
    pltpu.VMEM((4, GE), jnp.float32),
    pltpu.VMEM((4, GE), jnp.float32),
    pltpu.VMEM((4, GE), jnp.float32),
    pltpu.VMEM((3, D), jnp.float32),
]

_SC_SEMS = [pltpu.SemaphoreType.DMA] * 10

_sc_edge_deg = pl.kernel(
    functools.partial(_sc_edge_body, True),
    out_type=(jax.ShapeDtypeStruct((N, D), jnp.float32),
              jax.ShapeDtypeStruct((N, D), jnp.float32),
              jax.ShapeDtypeStruct((N,), jnp.float32),
              jax.ShapeDtypeStruct((N,), jnp.float32)),
    mesh=_SC_MESH,
    scratch_types=_SC_SCRATCH_COMMON + [
        pltpu.VMEM((RW,), jnp.float32),
        pltpu.VMEM((640,), jnp.float32),
        pltpu.VMEM_SHARED((N, D), jnp.float32),
        pltpu.VMEM_SHARED((N,), jnp.float32),
    ] + _SC_SEMS,
    compiler_params=_SC_PARAMS,
    name="sc_edge_deg",
)

_sc_edge = pl.kernel(
    functools.partial(_sc_edge_body, False),
    out_type=(jax.ShapeDtypeStruct((N, D), jnp.float32),
              jax.ShapeDtypeStruct((N, D), jnp.float32)),
    mesh=_SC_MESH,
    scratch_types=_SC_SCRATCH_COMMON + [
        pltpu.VMEM_SHARED((N, D), jnp.float32),
    ] + _SC_SEMS,
    compiler_params=_SC_PARAMS,
    name="sc_edge",
)


def _prep_body(gate_ref, emb_ref, w1aT_ref, h0_ref, p0_ref):
  g = gate_ref[...]
  iota = lax.broadcasted_iota(jnp.int32, (NB, F), 1)
  oh = jnp.where(g == iota, 1.0, 0.0).astype(jnp.float32)
  h0 = _dot(oh, emb_ref[...])
  h0_ref[...] = h0
  p0_ref[...] = _dot(h0, w1aT_ref[...])


_prep = pl.pallas_call(
    _prep_body,
    grid=(GRID_N,),
    in_specs=[
        pl.BlockSpec((NB, 1), lambda i: (i, 0)),
        pl.BlockSpec((F, F), lambda i: (0, 0)),
        pl.BlockSpec((F, D), lambda i: (0, 0)),
    ],
    out_specs=(pl.BlockSpec((NB, F), lambda i: (i, 0)),
               pl.BlockSpec((NB, D), lambda i: (i, 0))),
    out_shape=(jax.ShapeDtypeStruct((N, F), jnp.float32),
               jax.ShapeDtypeStruct((N, D), jnp.float32)),
)


def _rdeg_body(d0_ref, d1_ref, rd_ref):
  rd_ref[...] = 1.0 / jnp.maximum(d0_ref[...] + d1_ref[...], 1.0)


_rdeg = pl.pallas_call(
    _rdeg_body,
    grid=(GRID_N,),
    in_specs=[
        pl.BlockSpec((NB, 1), lambda i: (i, 0)),
        pl.BlockSpec((NB, 1), lambda i: (i, 0)),
    ],
    out_specs=pl.BlockSpec((NB, 1), lambda i: (i, 0)),
    out_shape=jax.ShapeDtypeStruct((N, 1), jnp.float32),
)


def _pre_body(h_ref, w2aT_ref, b2_ref, z_ref):
  z_ref[...] = _dot(h_ref[...], w2aT_ref[...]) + b2_ref[...]


_pre = pl.pallas_call(
    _pre_body,
    grid=(GRID_N,),
    in_specs=[
        pl.BlockSpec((NB, F), lambda i: (i, 0)),
        pl.BlockSpec((F, F), lambda i: (0, 0)),
        pl.BlockSpec((1, F), lambda i: (0, 0)),
    ],
    out_specs=pl.BlockSpec((NB, F), lambda i: (i, 0)),
    out_shape=jax.ShapeDtypeStruct((N, F), jnp.float32),
)


def _post_body(z_ref, a0_ref, a1_ref, rd_ref, w2bT_ref, w1aT_ref,
               hout_ref, pout_ref):
  hN = (a0_ref[...] + a1_ref[...]) * rd_ref[...]
  hn = jnp.maximum(z_ref[...] + _dot(hN, w2bT_ref[...]), 0.0)
  hout_ref[...] = hn
  pout_ref[...] = _dot(hn, w1aT_ref[...])


_post = pl.pallas_call(
    _post_body,
    grid=(GRID_N,),
    in_specs=[
        pl.BlockSpec((NB, F), lambda i: (i, 0)),
        pl.BlockSpec((NB, D), lambda i: (i, 0)),
        pl.BlockSpec((NB, D), lambda i: (i, 0)),
        pl.BlockSpec((NB, 1), lambda i: (i, 0)),
        pl.BlockSpec((D, F), lambda i: (0, 0)),
        pl.BlockSpec((F, D), lambda i: (0, 0)),
    ],
    out_specs=(pl.BlockSpec((NB, F), lambda i: (i, 0)),
               pl.BlockSpec((NB, D), lambda i: (i, 0))),
    out_shape=(jax.ShapeDtypeStruct((N, F), jnp.float32),
               jax.ShapeDtypeStruct((N, D), jnp.float32)),
)


def _final_body(z_ref, a0_ref, a1_ref, rd_ref, w2bT_ref,
                l1wT_ref, l1b_ref, l2wT_ref, l2b_ref, out_ref):
  hN = (a0_ref[...] + a1_ref[...]) * rd_ref[...]
  hn = jnp.maximum(z_ref[...] + _dot(hN, w2bT_ref[...]), 0.0)
  h1 = jnp.maximum(_dot(hn, l1wT_ref[...]) + l1b_ref[...], 0.0)
  out_ref[...] = _dot(h1, l2wT_ref[...]) + l2b_ref[...]


_final = pl.pallas_call(
    _final_body,
    grid=(GRID_N,),
    in_specs=[
        pl.BlockSpec((NB, F), lambda i: (i, 0)),
        pl.BlockSpec((NB, D), lambda i: (i, 0)),
        pl.BlockSpec((NB, D), lambda i: (i, 0)),
        pl.BlockSpec((NB, 1), lambda i: (i, 0)),
        pl.BlockSpec((D, F), lambda i: (0, 0)),
        pl.BlockSpec((F, F), lambda i: (0, 0)),
        pl.BlockSpec((1, F), lambda i: (0, 0)),
        pl.BlockSpec((F, D), lambda i: (0, 0)),
        pl.BlockSpec((1, D), lambda i: (0, 0)),
    ],
    out_specs=pl.BlockSpec((NB, D), lambda i: (i, 0)),
    out_shape=jax.ShapeDtypeStruct((N, D), jnp.float32),
)


def kernel(gate_type, edge_index, edge_src_idx, edge_dst_idx, edge_reversed,
           emb, W1_0, W2_0, b2_0, W1_1, W2_1, b2_1, W1_2, W2_2, b2_2,
           W1_3, W2_3, b2_3, W1_4, W2_4, b2_4, L1W, L1b, L2W, L2b):
  gate2d = gate_type.astype(jnp.int32).reshape(N, 1)
  src2d = edge_index[0].astype(jnp.int32).reshape(ROWS, RW)
  dst2d = edge_index[1].astype(jnp.int32).reshape(ROWS, RW)

  convs = [(W1_0, W2_0, b2_0), (W1_1, W2_1, b2_1), (W1_2, W2_2, b2_2),
           (W1_3, W2_3, b2_3), (W1_4, W2_4, b2_4)]
  W1aT, W1bT, W2aT, W2bT, b2r = [], [], [], [], []
  for W1, W2, b2 in convs:
    W1aT.append(W1[:, :F].T)
    W1bT.append(W1[:, F:F + 3].T)
    W2aT.append(W2[:, :F].T)
    W2bT.append(W2[:, F:].T)
    b2r.append(b2.reshape(1, F))

  h, p = _prep(gate2d, emb, W1aT[0])

  out = None
  rdeg = None
  for l in range(5):
    if l == 0:
      a0, a1, deg0, deg1 = _sc_edge_deg(
          p, edge_src_idx, edge_dst_idx, edge_reversed, W1bT[l], src2d, dst2d)
      rdeg = _rdeg(deg0.reshape(N, 1), deg1.reshape(N, 1))
    else:
      a0, a1 = _sc_edge(
          p, edge_src_idx, edge_dst_idx, edge_reversed, W1bT[l], src2d, dst2d)
    z = _pre(h, W2aT[l], b2r[l])
    if l < 4:
      h, p = _post(z, a0, a1, rdeg, W2bT[l], W1aT[l + 1])
    else:
      out = _final(z, a0, a1, rdeg, W2bT[4],
                   L1W.T, L1b.reshape(1, F), L2W.T, L2b.reshape(1, D))
  return out
